# Initial kernel scaffold; baseline (speedup 1.0000x reference)
import jax
import jax.numpy as jnp
from jax.experimental import pallas as pl


def kernel(features, edge_index0, edge_index1, edge_index2, P0, P1, W1, b1, W2, b2):
    raise NotImplementedError("write your pallas kernel here")


# speedup vs baseline: 5.4981x; 5.4981x over previous
import jax
import jax.numpy as jnp
from jax import lax
from jax.experimental import pallas as pl
from jax.experimental.pallas import tpu as pltpu
from jax.experimental.pallas import tpu_sc as plsc

NC = 2
NS = 16
NW = NC * NS
EB = 128
NBUF = 4

N0, N1, N2 = 10000, 5000, 2500
NP0, NP1, NP2 = 10112, 5120, 2560
NB0, NB1, NB2 = 80, 40, 20
S0, S1, S2 = 10240, 6144, 4096

_MESH = plsc.VectorSubcoreMesh(
    core_axis_name="c", subcore_axis_name="s", num_cores=NC, num_subcores=NS)


def _zero_vec(ref, n):
  @pl.loop(0, n // 16)
  def _(i):
    ref[pl.ds(i * 16, 16)] = jnp.zeros((16,), jnp.float32)


def _deg_body(e0s, e0d, e1s, e1d, e2s, e2d,
              o0s, o0d, o1s, o1d, o2s, o2d,
              idxv, onesv, zbuf,
              sec0s, sec0d, sec1s, sec1d, sec2s, sec2d, dsem):
  cid = lax.axis_index("c")
  sid = lax.axis_index("s")
  wid = sid * NC + cid

  _zero_vec(zbuf, 640)
  @pl.loop(0, EB // 16)
  def _(i):
    onesv[pl.ds(i * 16, 16)] = jnp.ones((16,), jnp.float32)

  passes = [(e0s, sec0s, o0s, S0, NB0), (e0d, sec0d, o0d, S0, NB0),
            (e1s, sec1s, o1s, S1, NB1), (e1d, sec1d, o1d, S1, NB1),
            (e2s, sec2s, o2s, S2, NB2), (e2d, sec2d, o2d, S2, NB2)]

  for _, sec, _, size, _ in passes:
    sz = size // NS
    pltpu.sync_copy(zbuf.at[pl.ds(0, sz)], sec.at[pl.ds(sid * sz, sz)])
  plsc.subcore_barrier()

  for e_ref, sec, _, _, nb in passes:
    pltpu.sync_copy(e_ref.at[wid], idxv.at[pl.ds(0, nb)])
    @pl.loop(0, nb // 4)
    def _(g):
      descs = []
      for b in range(4):
        descs.append(
            pltpu.async_copy(onesv, sec.at[idxv.at[g * 4 + b]], dsem,
                             add=True))
      for d_ in descs:
        d_.wait()
  plsc.subcore_barrier()

  for _, sec, out, size, _ in passes:
    sz = size // NS
    off = pl.multiple_of(cid * size + sid * sz, 128)
    pltpu.sync_copy(sec.at[pl.ds(sid * sz, sz)], out.at[pl.ds(off, sz)])


_deg_kernel = pl.kernel(
    _deg_body,
    out_type=[jax.ShapeDtypeStruct((NC * S0,), jnp.float32),
              jax.ShapeDtypeStruct((NC * S0,), jnp.float32),
              jax.ShapeDtypeStruct((NC * S1,), jnp.float32),
              jax.ShapeDtypeStruct((NC * S1,), jnp.float32),
              jax.ShapeDtypeStruct((NC * S2,), jnp.float32),
              jax.ShapeDtypeStruct((NC * S2,), jnp.float32)],
    mesh=_MESH,
    scratch_types=[
        pltpu.VMEM((NB0, EB), jnp.int32),
        pltpu.VMEM((EB,), jnp.float32),
        pltpu.VMEM((640,), jnp.float32),
        pltpu.VMEM_SHARED((S0,), jnp.float32),
        pltpu.VMEM_SHARED((S0,), jnp.float32),
        pltpu.VMEM_SHARED((S1,), jnp.float32),
        pltpu.VMEM_SHARED((S1,), jnp.float32),
        pltpu.VMEM_SHARED((S2,), jnp.float32),
        pltpu.VMEM_SHARED((S2,), jnp.float32),
        pltpu.SemaphoreType.DMA,
    ])


def _make_prop(npad, d, nb):
  ngroups = nb // NBUF
  rows_per_tile = npad // NS

  def body(xs_hbm, src_hbm, dst_hbm, out_hbm, srcv, dstv, *rest):
    rows = list(rest[0:NBUF])
    acc = rest[NBUF]
    gsem = list(rest[NBUF + 1: NBUF + 1 + NBUF])
    ssem = list(rest[NBUF + 1 + NBUF:])
    cid = lax.axis_index("c")
    sid = lax.axis_index("s")
    wid = sid * NC + cid

    pltpu.sync_copy(src_hbm.at[wid], srcv)
    pltpu.sync_copy(dst_hbm.at[wid], dstv)

    @pl.loop(0, EB)
    def _(r):
      @pl.loop(0, d // 16)
      def _(c):
        rows[0][r, pl.ds(c * 16, 16)] = jnp.zeros((16,), jnp.float32)
    base = pl.multiple_of(sid * rows_per_tile, 8)
    off = 0
    while off < rows_per_tile:
      c = min(EB, rows_per_tile - off)
      pltpu.sync_copy(rows[0].at[pl.ds(0, c)], acc.at[pl.ds(base + off, c)])
      off += c
    plsc.subcore_barrier()

    def issue_gather(j, b):
      return pltpu.async_copy(xs_hbm.at[srcv.at[j]], rows[b], gsem[b])

    def issue_scatter(j, b):
      return pltpu.async_copy(rows[b], acc.at[dstv.at[j]], ssem[b], add=True)

    gds = [issue_gather(b, b) for b in range(NBUF)]

    @pl.loop(0, ngroups - 1)
    def _(g):
      sds = []
      for b in range(NBUF):
        gds[b].wait()
        sds.append(issue_scatter(g * NBUF + b, b))
      for b in range(NBUF):
        sds[b].wait()
        issue_gather((g + 1) * NBUF + b, b)

    last = ngroups - 1
    sds = []
    for b in range(NBUF):
      gds[b].wait()
      sds.append(issue_scatter(last * NBUF + b, b))
    for b in range(NBUF):
      sds[b].wait()
    plsc.subcore_barrier()

    off = 0
    while off < rows_per_tile:
      c = min(512, rows_per_tile - off)
      pltpu.sync_copy(acc.at[pl.ds(base + off, c)],
                      out_hbm.at[cid, pl.ds(base + off, c)])
      off += c

  return pl.kernel(
      body,
      out_type=jax.ShapeDtypeStruct((NC, npad, d), jnp.float32),
      mesh=_MESH,
      compiler_params=pltpu.CompilerParams(use_tc_tiling_on_sc=False),
      scratch_types=(
          [pltpu.VMEM((nb, EB), jnp.int32) for _ in range(2)]
          + [pltpu.VMEM((EB, d), jnp.float32) for _ in range(NBUF)]
          + [pltpu.VMEM_SHARED((npad, d), jnp.float32)]
          + [pltpu.SemaphoreType.DMA for _ in range(2 * NBUF)]))


_prop0h = _make_prop(NP0, 64, NB0)
_prop1 = _make_prop(NP1, 128, NB1)
_prop2 = _make_prop(NP2, 64, NB2)


def _tc_invs(a, b, c, d, n, blk):
  def body(a_r, b_r, c_r, d_r, ns_o, nd_o):
    def invs(x, y):
      dg = x[...] + y[...]
      return jnp.where(dg > 0, lax.rsqrt(dg), 0.0)
    ns_o[...] = invs(a_r, b_r)
    nd_o[...] = invs(c_r, d_r)

  spec = pl.BlockSpec((blk, 1), lambda i: (i, 0))
  return pl.pallas_call(
      body,
      grid=(n // blk,),
      in_specs=[spec, spec, spec, spec],
      out_specs=[spec, spec],
      out_shape=[jax.ShapeDtypeStruct((n, 1), jnp.float32),
                 jax.ShapeDtypeStruct((n, 1), jnp.float32)],
  )(a, b, c, d)


def _tc_scale0(featp, ns0):
  blk = 1264

  def body(f_r, ns_r, xlo_o, xhi_o):
    xs = f_r[...] * ns_r[...]
    xlo_o[...] = xs[:, :64]
    xhi_o[...] = xs[:, 64:]

  return pl.pallas_call(
      body,
      grid=(NP0 // blk,),
      in_specs=[pl.BlockSpec((blk, 128), lambda i: (i, 0)),
                pl.BlockSpec((blk, 1), lambda i: (i, 0))],
      out_specs=[pl.BlockSpec((blk, 64), lambda i: (i, 0)),
                 pl.BlockSpec((blk, 64), lambda i: (i, 0))],
      out_shape=[jax.ShapeDtypeStruct((NP0, 64), jnp.float32),
                 jax.ShapeDtypeStruct((NP0, 64), jnp.float32)],
  )(featp, ns0)


def _tc_layer0(pal, pbl, pah, pbh, nd0, W1a, W1b, b1r):
  def body(pal_r, pbl_r, pah_r, pbh_r, nd_r, wa_r, wb_r, b_r, z_r):
    nd = nd_r[...]
    hl = (pal_r[...] + pbl_r[...]) * nd
    hh = (pah_r[...] + pbh_r[...]) * nd
    acc = (jnp.dot(hl, wa_r[...], preferred_element_type=jnp.float32)
           + jnp.dot(hh, wb_r[...], preferred_element_type=jnp.float32))
    z_r[...] = jnp.maximum(acc + b_r[...], 0.0)

  return pl.pallas_call(
      body,
      grid=(10,),
      in_specs=[pl.BlockSpec((1000, 64), lambda i: (i, 0)),
                pl.BlockSpec((1000, 64), lambda i: (i, 0)),
                pl.BlockSpec((1000, 64), lambda i: (i, 0)),
                pl.BlockSpec((1000, 64), lambda i: (i, 0)),
                pl.BlockSpec((1000, 1), lambda i: (i, 0)),
                pl.BlockSpec((64, 128), lambda i: (0, 0)),
                pl.BlockSpec((64, 128), lambda i: (0, 0)),
                pl.BlockSpec((1, 128), lambda i: (0, 0))],
      out_specs=pl.BlockSpec((1000, 128), lambda i: (i, 0)),
      out_shape=jax.ShapeDtypeStruct((N0, 128), jnp.float32),
  )(pal, pbl, pah, pbh, nd0, W1a, W1b, b1r)


def _tc_proj0(z, P0, ns1):
  nk = 10

  def body(p_r, z_r, ns_r, emb_r, xs_r):
    k = pl.program_id(1)
    part = lax.dot_general(p_r[...], z_r[...], (((0,), (0,)), ((), ())),
                           preferred_element_type=jnp.float32)
    @pl.when(k == 0)
    def _():
      emb_r[...] = part
    @pl.when(k > 0)
    def _():
      emb_r[...] += part
    @pl.when(k == nk - 1)
    def _():
      xs_r[...] = emb_r[...] * ns_r[...]

  return pl.pallas_call(
      body,
      grid=(10, nk),
      in_specs=[pl.BlockSpec((1000, 512), lambda j, k: (k, j)),
                pl.BlockSpec((1000, 128), lambda j, k: (k, 0)),
                pl.BlockSpec((512, 1), lambda j, k: (j, 0))],
      out_specs=[pl.BlockSpec((512, 128), lambda j, k: (j, 0)),
                 pl.BlockSpec((512, 128), lambda j, k: (j, 0))],
      out_shape=[jax.ShapeDtypeStruct((N1, 128), jnp.float32),
                 jax.ShapeDtypeStruct((N1, 128), jnp.float32)],
  )(P0, z, ns1)


def _tc_mid(pa, pb, ns1, nd1):
  def body(pa_r, pb_r, ns_r, nd_r, o_r):
    o_r[...] = (pa_r[...] + pb_r[...]) * (ns_r[...] * nd_r[...])

  return pl.pallas_call(
      body,
      out_shape=jax.ShapeDtypeStruct((N1, 128), jnp.float32),
  )(pa, pb, ns1, nd1)


def _tc_proj1(pa, pb, nd1, P1, W2, ns2):
  nk = 5

  def body(p_r, pa_r, pb_r, nd_r, w2_r, ns_r, x2_r, acc_r):
    k = pl.program_id(1)
    h1 = (pa_r[...] + pb_r[...]) * nd_r[...]
    part = lax.dot_general(p_r[...], h1, (((0,), (0,)), ((), ())),
                           preferred_element_type=jnp.float32)
    @pl.when(k == 0)
    def _():
      acc_r[...] = part
    @pl.when(k > 0)
    def _():
      acc_r[...] += part
    @pl.when(k == nk - 1)
    def _():
      x2_r[...] = jnp.dot(acc_r[...], w2_r[...],
                          preferred_element_type=jnp.float32) * ns_r[...]

  return pl.pallas_call(
      body,
      grid=(5, nk),
      in_specs=[pl.BlockSpec((1000, 512), lambda j, k: (k, j)),
                pl.BlockSpec((1000, 128), lambda j, k: (k, 0)),
                pl.BlockSpec((1000, 128), lambda j, k: (k, 0)),
                pl.BlockSpec((1000, 1), lambda j, k: (k, 0)),
                pl.BlockSpec((128, 40), lambda j, k: (0, 0)),
                pl.BlockSpec((512, 1), lambda j, k: (j, 0))],
      out_specs=pl.BlockSpec((512, 40), lambda j, k: (j, 0)),
      out_shape=jax.ShapeDtypeStruct((N2, 40), jnp.float32),
      scratch_shapes=[pltpu.VMEM((512, 128), jnp.float32)],
  )(P1, pa, pb, nd1, W2, ns2)


def _tc_final(pa, pb, nd2, b2r):
  def body(pa_r, pb_r, nd_r, b_r, o_r):
    o_r[...] = (pa_r[...] + pb_r[...]) * nd_r[...] + b_r[...]

  return pl.pallas_call(
      body,
      out_shape=jax.ShapeDtypeStruct((N2, 40), jnp.float32),
  )(pa, pb, nd2, b2r)


def _prep_edges(ei, nb, dummy):
  e_pad = NW * nb * EB
  pad = e_pad - ei.shape[1]
  fill = jnp.full((pad,), dummy, dtype=jnp.int32)
  src = jnp.concatenate([ei[0], fill]).reshape(NW, nb, EB)
  dst = jnp.concatenate([ei[1], fill]).reshape(NW, nb, EB)
  return src, dst


def kernel(features, edge_index0, edge_index1, edge_index2, P0, P1, W1, b1,
           W2, b2):
  s0, d0 = _prep_edges(edge_index0, NB0, N0)
  s1, d1 = _prep_edges(edge_index1, NB1, N1)
  s2, d2 = _prep_edges(edge_index2, NB2, N2)

  dg0s, dg0d, dg1s, dg1d, dg2s, dg2d = _deg_kernel(s0, d0, s1, d1, s2, d2)

  featp = jnp.pad(features, ((0, NP0 - N0), (0, 0)))

  def cut(dg, size, npad):
    return dg[:npad, None], dg[size:size + npad, None]

  a0s, b0s = cut(dg0s, S0, NP0)
  a0d, b0d = cut(dg0d, S0, NP0)
  a1s, b1s = cut(dg1s, S1, NP1)
  a1d, b1d = cut(dg1d, S1, NP1)
  a2s, b2s = cut(dg2s, S2, NP2)
  a2d, b2d = cut(dg2d, S2, NP2)

  ns0, nd0 = _tc_invs(a0s, b0s, a0d, b0d, NP0, 1264)
  ns1, nd1 = _tc_invs(a1s, b1s, a1d, b1d, NP1, 640)
  ns2, nd2 = _tc_invs(a2s, b2s, a2d, b2d, NP2, 320)
  xlo, xhi = _tc_scale0(featp, ns0)

  part0a = _prop0h(xlo, s0, d0)
  part0b = _prop0h(xhi, s0, d0)
  z = _tc_layer0(part0a[0, :N0], part0a[1, :N0], part0b[0, :N0],
                 part0b[1, :N0], nd0[:N0], W1[:64], W1[64:],
                 b1.reshape(1, 128))

  emb, xs1 = _tc_proj0(z, P0, ns1[:N1])

  part1 = _prop1(jnp.pad(xs1, ((0, NP1 - N1), (0, 0))), s1, d1)
  xmid = _tc_mid(part1[0, :N1], part1[1, :N1], ns1[:N1], nd1[:N1])

  part1b = _prop1(jnp.pad(xmid, ((0, NP1 - N1), (0, 0))), s1, d1)
  x2s = _tc_proj1(part1b[0, :N1], part1b[1, :N1], nd1[:N1], P1, W2,
                  ns2[:N2])

  part2 = _prop2(jnp.pad(x2s, ((0, NP2 - N2), (0, 24))), s2, d2)
  out = _tc_final(part2[0, :N2, :40], part2[1, :N2, :40], nd2[:N2],
                  b2.reshape(1, 40))
  return (out, emb)


# speedup vs baseline: 8.2191x; 1.4949x over previous
import jax
import jax.numpy as jnp
from jax import lax
from jax.experimental import pallas as pl
from jax.experimental.pallas import tpu as pltpu
from jax.experimental.pallas import tpu_sc as plsc

NC = 2
NS = 16
NW = NC * NS
EB = 128
NBUF = 4

N0, N1, N2 = 10000, 5000, 2500
NP0, NP1, NP2 = 10112, 5120, 2560
NB0, NB1, NB2 = 80, 40, 20
NBS0, NBS1 = 160, 80
S0, S1, S2 = 10240, 6144, 4096

_MESH = plsc.VectorSubcoreMesh(
    core_axis_name="c", subcore_axis_name="s", num_cores=NC, num_subcores=NS)


def _zero_vec(ref, n):
  @pl.loop(0, n // 16)
  def _(i):
    ref[pl.ds(i * 16, 16)] = jnp.zeros((16,), jnp.float32)


def _deg_body(e0s, e0d, e1s, e1d, e2s, e2d,
              o0s, o0d, o1s, o1d, o2s, o2d,
              idxv, onesv, zbuf,
              sec0s, sec0d, sec1s, sec1d, sec2s, sec2d, dsem):
  cid = lax.axis_index("c")
  sid = lax.axis_index("s")
  wid = sid * NC + cid

  _zero_vec(zbuf, 640)
  @pl.loop(0, EB // 16)
  def _(i):
    onesv[pl.ds(i * 16, 16)] = jnp.ones((16,), jnp.float32)

  passes = [(e0s, sec0s, o0s, S0, NB0), (e0d, sec0d, o0d, S0, NB0),
            (e1s, sec1s, o1s, S1, NB1), (e1d, sec1d, o1d, S1, NB1),
            (e2s, sec2s, o2s, S2, NB2), (e2d, sec2d, o2d, S2, NB2)]

  for _, sec, _, size, _ in passes:
    sz = size // NS
    pltpu.sync_copy(zbuf.at[pl.ds(0, sz)], sec.at[pl.ds(sid * sz, sz)])
  plsc.subcore_barrier()

  for e_ref, sec, _, _, nb in passes:
    pltpu.sync_copy(e_ref.at[wid], idxv.at[pl.ds(0, nb)])
    @pl.loop(0, nb // 4)
    def _(g):
      descs = []
      for b in range(4):
        descs.append(
            pltpu.async_copy(onesv, sec.at[idxv.at[g * 4 + b]], dsem,
                             add=True))
      for d_ in descs:
        d_.wait()
  plsc.subcore_barrier()

  for _, sec, out, size, _ in passes:
    sz = size // NS
    off = pl.multiple_of(cid * size + sid * sz, 128)
    pltpu.sync_copy(sec.at[pl.ds(sid * sz, sz)], out.at[pl.ds(off, sz)])


_deg_kernel = pl.kernel(
    _deg_body,
    out_type=[jax.ShapeDtypeStruct((NC * S0,), jnp.float32),
              jax.ShapeDtypeStruct((NC * S0,), jnp.float32),
              jax.ShapeDtypeStruct((NC * S1,), jnp.float32),
              jax.ShapeDtypeStruct((NC * S1,), jnp.float32),
              jax.ShapeDtypeStruct((NC * S2,), jnp.float32),
              jax.ShapeDtypeStruct((NC * S2,), jnp.float32)],
    mesh=_MESH,
    scratch_types=[
        pltpu.VMEM((NB0, EB), jnp.int32),
        pltpu.VMEM((EB,), jnp.float32),
        pltpu.VMEM((640,), jnp.float32),
        pltpu.VMEM_SHARED((S0,), jnp.float32),
        pltpu.VMEM_SHARED((S0,), jnp.float32),
        pltpu.VMEM_SHARED((S1,), jnp.float32),
        pltpu.VMEM_SHARED((S1,), jnp.float32),
        pltpu.VMEM_SHARED((S2,), jnp.float32),
        pltpu.VMEM_SHARED((S2,), jnp.float32),
        pltpu.SemaphoreType.DMA,
    ])


def _zero_acc_slice(rows0, acc, d, base, rows_per_tile):
  @pl.loop(0, EB)
  def _(r):
    @pl.loop(0, d // 16)
    def _(c):
      rows0[r, pl.ds(c * 16, 16)] = jnp.zeros((16,), jnp.float32)
  off = 0
  while off < rows_per_tile:
    c = min(EB, rows_per_tile - off)
    pltpu.sync_copy(rows0.at[pl.ds(0, c)], acc.at[pl.ds(base + off, c)])
    off += c


def _ring(ngroups, srcv, dstv, rows, gsem, ssem, gather_from, acc):
  def issue_gather(j, b):
    return pltpu.async_copy(gather_from.at[srcv.at[j]], rows[b], gsem[b])

  def issue_scatter(j, b):
    return pltpu.async_copy(rows[b], acc.at[dstv.at[j]], ssem[b], add=True)

  gds = [issue_gather(b, b) for b in range(NBUF)]

  @pl.loop(0, ngroups - 1)
  def _(g):
    sds = []
    for b in range(NBUF):
      gds[b].wait()
      sds.append(issue_scatter(g * NBUF + b, b))
    for b in range(NBUF):
      sds[b].wait()
      issue_gather((g + 1) * NBUF + b, b)

  last = ngroups - 1
  sds = []
  for b in range(NBUF):
    gds[b].wait()
    sds.append(issue_scatter(last * NBUF + b, b))
  for b in range(NBUF):
    sds[b].wait()


def _make_prop0():
  d = 64
  nbs = NBS0
  ngroups = nbs // NBUF
  rows_per_tile = NP0 // NS

  def body(xs_hbm, src_hbm, dst_hbm, out_hbm, srcv, dstv, *rest):
    rows = list(rest[0:NBUF])
    acc = rest[NBUF]
    gsem = list(rest[NBUF + 1: NBUF + 1 + NBUF])
    ssem = list(rest[NBUF + 1 + NBUF:])
    cid = lax.axis_index("c")
    sid = lax.axis_index("s")

    pltpu.sync_copy(src_hbm.at[cid * NS + sid], srcv)
    pltpu.sync_copy(dst_hbm.at[sid], dstv)

    base = pl.multiple_of(sid * rows_per_tile, 8)
    _zero_acc_slice(rows[0], acc, d, base, rows_per_tile)
    plsc.subcore_barrier()

    _ring(ngroups, srcv, dstv, rows, gsem, ssem, xs_hbm, acc)
    plsc.subcore_barrier()

    off = 0
    while off < rows_per_tile:
      c = min(512, rows_per_tile - off)
      pltpu.sync_copy(acc.at[pl.ds(base + off, c)],
                      out_hbm.at[cid, pl.ds(base + off, c)])
      off += c

  return pl.kernel(
      body,
      out_type=jax.ShapeDtypeStruct((NC, NP0, d), jnp.float32),
      mesh=_MESH,
      compiler_params=pltpu.CompilerParams(use_tc_tiling_on_sc=False),
      scratch_types=(
          [pltpu.VMEM((nbs, EB), jnp.int32) for _ in range(2)]
          + [pltpu.VMEM((EB, d), jnp.float32) for _ in range(NBUF)]
          + [pltpu.VMEM_SHARED((NP0, d), jnp.float32)]
          + [pltpu.SemaphoreType.DMA for _ in range(2 * NBUF)]))


def _make_prop1():
  d = 64
  nbs = NBS1
  ngroups = nbs // NBUF
  rows_per_tile = NP1 // NS

  def body(xs_hbm, src_hbm, dst_hbm, out_hbm, srcv, dstv, *rest):
    rows = list(rest[0:NBUF])
    table = rest[NBUF]
    acc = rest[NBUF + 1]
    gsem = list(rest[NBUF + 2: NBUF + 2 + NBUF])
    ssem = list(rest[NBUF + 2 + NBUF:])
    cid = lax.axis_index("c")
    sid = lax.axis_index("s")

    pltpu.sync_copy(src_hbm.at[sid], srcv)
    pltpu.sync_copy(dst_hbm.at[sid], dstv)

    base = pl.multiple_of(sid * rows_per_tile, 8)
    hoff = pl.multiple_of(cid * NP1 + sid * rows_per_tile, 8)
    pltpu.sync_copy(xs_hbm.at[pl.ds(hoff, rows_per_tile)],
                    table.at[pl.ds(base, rows_per_tile)])
    _zero_acc_slice(rows[0], acc, d, base, rows_per_tile)
    plsc.subcore_barrier()

    _ring(ngroups, srcv, dstv, rows, gsem, ssem, table, acc)
    plsc.subcore_barrier()

    off = 0
    while off < rows_per_tile:
      c = min(512, rows_per_tile - off)
      pltpu.sync_copy(acc.at[pl.ds(base + off, c)],
                      out_hbm.at[cid, pl.ds(base + off, c)])
      off += c

  return pl.kernel(
      body,
      out_type=jax.ShapeDtypeStruct((NC, NP1, d), jnp.float32),
      mesh=_MESH,
      compiler_params=pltpu.CompilerParams(use_tc_tiling_on_sc=False),
      scratch_types=(
          [pltpu.VMEM((nbs, EB), jnp.int32) for _ in range(2)]
          + [pltpu.VMEM((EB, d), jnp.float32) for _ in range(NBUF)]
          + [pltpu.VMEM_SHARED((NP1, d), jnp.float32)]
          + [pltpu.VMEM_SHARED((NP1, d), jnp.float32)]
          + [pltpu.SemaphoreType.DMA for _ in range(2 * NBUF)]))


def _make_prop2():
  d = 64
  nb = NB2
  ngroups = nb // NBUF
  rows_per_tile = NP2 // NS

  def body(xs_hbm, src_hbm, dst_hbm, out_hbm, srcv, dstv, *rest):
    rows = list(rest[0:NBUF])
    table = rest[NBUF]
    acc = rest[NBUF + 1]
    gsem = list(rest[NBUF + 2: NBUF + 2 + NBUF])
    ssem = list(rest[NBUF + 2 + NBUF:])
    cid = lax.axis_index("c")
    sid = lax.axis_index("s")
    wid = sid * NC + cid

    pltpu.sync_copy(src_hbm.at[wid], srcv)
    pltpu.sync_copy(dst_hbm.at[wid], dstv)

    base = pl.multiple_of(sid * rows_per_tile, 8)
    pltpu.sync_copy(xs_hbm.at[pl.ds(base, rows_per_tile)],
                    table.at[pl.ds(base, rows_per_tile)])
    _zero_acc_slice(rows[0], acc, d, base, rows_per_tile)
    plsc.subcore_barrier()

    _ring(ngroups, srcv, dstv, rows, gsem, ssem, table, acc)
    plsc.subcore_barrier()

    off = 0
    while off < rows_per_tile:
      c = min(512, rows_per_tile - off)
      pltpu.sync_copy(acc.at[pl.ds(base + off, c)],
                      out_hbm.at[cid, pl.ds(base + off, c)])
      off += c

  return pl.kernel(
      body,
      out_type=jax.ShapeDtypeStruct((NC, NP2, d), jnp.float32),
      mesh=_MESH,
      compiler_params=pltpu.CompilerParams(use_tc_tiling_on_sc=False),
      scratch_types=(
          [pltpu.VMEM((nb, EB), jnp.int32) for _ in range(2)]
          + [pltpu.VMEM((EB, d), jnp.float32) for _ in range(NBUF)]
          + [pltpu.VMEM_SHARED((NP2, d), jnp.float32)]
          + [pltpu.VMEM_SHARED((NP2, d), jnp.float32)]
          + [pltpu.SemaphoreType.DMA for _ in range(2 * NBUF)]))


_prop0 = _make_prop0()
_prop1 = _make_prop1()
_prop2 = _make_prop2()


def _tc_invs(a, b, c, d, n, blk):
  def body(a_r, b_r, c_r, d_r, ns_o, nd_o):
    def invs(x, y):
      dg = x[...] + y[...]
      return jnp.where(dg > 0, lax.rsqrt(dg), 0.0)
    ns_o[...] = invs(a_r, b_r)
    nd_o[...] = invs(c_r, d_r)

  spec = pl.BlockSpec((blk, 1), lambda i: (i, 0))
  return pl.pallas_call(
      body,
      grid=(n // blk,),
      in_specs=[spec, spec, spec, spec],
      out_specs=[spec, spec],
      out_shape=[jax.ShapeDtypeStruct((n, 1), jnp.float32),
                 jax.ShapeDtypeStruct((n, 1), jnp.float32)],
  )(a, b, c, d)


def _tc_scale0(featp, ns0):
  blk = 1264

  def body(f_r, ns_r, xlo_o, xhi_o):
    xs = f_r[...] * ns_r[...]
    xlo_o[...] = xs[:, :64]
    xhi_o[...] = xs[:, 64:]

  return pl.pallas_call(
      body,
      grid=(NP0 // blk,),
      in_specs=[pl.BlockSpec((blk, 128), lambda i: (i, 0)),
                pl.BlockSpec((blk, 1), lambda i: (i, 0))],
      out_specs=[pl.BlockSpec((blk, 64), lambda i: (i, 0)),
                 pl.BlockSpec((blk, 64), lambda i: (i, 0))],
      out_shape=[jax.ShapeDtypeStruct((NP0, 64), jnp.float32),
                 jax.ShapeDtypeStruct((NP0, 64), jnp.float32)],
  )(featp, ns0)


def _tc_layer0(plo, phi, nd0, W1a, W1b, b1r):
  def body(plo_r, phi_r, nd_r, wa_r, wb_r, b_r, z_r):
    nd = nd_r[...]
    hl = plo_r[...] * nd
    hh = phi_r[...] * nd
    acc = (jnp.dot(hl, wa_r[...], preferred_element_type=jnp.float32)
           + jnp.dot(hh, wb_r[...], preferred_element_type=jnp.float32))
    z_r[...] = jnp.maximum(acc + b_r[...], 0.0)

  return pl.pallas_call(
      body,
      grid=(10,),
      in_specs=[pl.BlockSpec((1000, 64), lambda i: (i, 0)),
                pl.BlockSpec((1000, 64), lambda i: (i, 0)),
                pl.BlockSpec((1000, 1), lambda i: (i, 0)),
                pl.BlockSpec((64, 128), lambda i: (0, 0)),
                pl.BlockSpec((64, 128), lambda i: (0, 0)),
                pl.BlockSpec((1, 128), lambda i: (0, 0))],
      out_specs=pl.BlockSpec((1000, 128), lambda i: (i, 0)),
      out_shape=jax.ShapeDtypeStruct((N0, 128), jnp.float32),
  )(plo, phi, nd0, W1a, W1b, b1r)


def _tc_proj0(z, P0, ns1):
  nk = 10

  def body(p_r, z_r, ns_r, emb_r, xlo_r, xhi_r):
    k = pl.program_id(1)
    part = lax.dot_general(p_r[...], z_r[...], (((0,), (0,)), ((), ())),
                           preferred_element_type=jnp.float32)
    @pl.when(k == 0)
    def _():
      emb_r[...] = part
    @pl.when(k > 0)
    def _():
      emb_r[...] += part
    @pl.when(k == nk - 1)
    def _():
      xs = emb_r[...] * ns_r[...]
      xlo_r[...] = xs[:, :64]
      xhi_r[...] = xs[:, 64:]

  return pl.pallas_call(
      body,
      grid=(10, nk),
      in_specs=[pl.BlockSpec((1000, 512), lambda j, k: (k, j)),
                pl.BlockSpec((1000, 128), lambda j, k: (k, 0)),
                pl.BlockSpec((512, 1), lambda j, k: (j, 0))],
      out_specs=[pl.BlockSpec((512, 128), lambda j, k: (j, 0)),
                 pl.BlockSpec((512, 64), lambda j, k: (j, 0)),
                 pl.BlockSpec((512, 64), lambda j, k: (j, 0))],
      out_shape=[jax.ShapeDtypeStruct((N1, 128), jnp.float32),
                 jax.ShapeDtypeStruct((N1, 64), jnp.float32),
                 jax.ShapeDtypeStruct((N1, 64), jnp.float32)],
  )(P0, z, ns1)


def _tc_mid(xt, snd2):
  blk = 1024

  def body(x_r, s_r, o_r):
    o_r[...] = x_r[...] * s_r[...]

  return pl.pallas_call(
      body,
      grid=(2 * NP1 // blk,),
      in_specs=[pl.BlockSpec((blk, 64), lambda i: (i, 0)),
                pl.BlockSpec((blk, 1), lambda i: (i, 0))],
      out_specs=pl.BlockSpec((blk, 64), lambda i: (i, 0)),
      out_shape=jax.ShapeDtypeStruct((2 * NP1, 64), jnp.float32),
  )(xt, snd2)


def _tc_proj1(plo, phi, nd1, P1, W2, ns2):
  nk = 5

  def body(p_r, plo_r, phi_r, nd_r, w2_r, ns_r, x2_r, acc_r):
    k = pl.program_id(1)
    h1 = jnp.concatenate([plo_r[...], phi_r[...]], axis=1) * nd_r[...]
    part = lax.dot_general(p_r[...], h1, (((0,), (0,)), ((), ())),
                           preferred_element_type=jnp.float32)
    @pl.when(k == 0)
    def _():
      acc_r[...] = part
    @pl.when(k > 0)
    def _():
      acc_r[...] += part
    @pl.when(k == nk - 1)
    def _():
      x2_r[...] = jnp.dot(acc_r[...], w2_r[...],
                          preferred_element_type=jnp.float32) * ns_r[...]

  return pl.pallas_call(
      body,
      grid=(5, nk),
      in_specs=[pl.BlockSpec((1000, 512), lambda j, k: (k, j)),
                pl.BlockSpec((1000, 64), lambda j, k: (k, 0)),
                pl.BlockSpec((1000, 64), lambda j, k: (k, 0)),
                pl.BlockSpec((1000, 1), lambda j, k: (k, 0)),
                pl.BlockSpec((128, 40), lambda j, k: (0, 0)),
                pl.BlockSpec((512, 1), lambda j, k: (j, 0))],
      out_specs=pl.BlockSpec((512, 40), lambda j, k: (j, 0)),
      out_shape=jax.ShapeDtypeStruct((N2, 40), jnp.float32),
      scratch_shapes=[pltpu.VMEM((512, 128), jnp.float32)],
  )(P1, plo, phi, nd1, W2, ns2)


def _tc_final(pa, pb, nd2, b2r):
  def body(pa_r, pb_r, nd_r, b_r, o_r):
    o_r[...] = (pa_r[...] + pb_r[...]) * nd_r[...] + b_r[...]

  return pl.pallas_call(
      body,
      out_shape=jax.ShapeDtypeStruct((N2, 40), jnp.float32),
  )(pa, pb, nd2, b2r)


def _prep_edges(ei, nb, dummy):
  e_pad = NW * nb * EB
  pad = e_pad - ei.shape[1]
  fill = jnp.full((pad,), dummy, dtype=jnp.int32)
  src = jnp.concatenate([ei[0], fill]).reshape(NW, nb, EB)
  dst = jnp.concatenate([ei[1], fill]).reshape(NW, nb, EB)
  return src, dst


def kernel(features, edge_index0, edge_index1, edge_index2, P0, P1, W1, b1,
           W2, b2):
  s0, d0 = _prep_edges(edge_index0, NB0, N0)
  s1, d1 = _prep_edges(edge_index1, NB1, N1)
  s2, d2 = _prep_edges(edge_index2, NB2, N2)

  dg0s, dg0d, dg1s, dg1d, dg2s, dg2d = _deg_kernel(s0, d0, s1, d1, s2, d2)

  s0s = s0.reshape(NS, NBS0, EB)
  s0c = jnp.concatenate([s0s, s0s + NP0], axis=0)
  d0c = d0.reshape(NS, NBS0, EB)
  s1c = s1.reshape(NS, NBS1, EB)
  d1c = d1.reshape(NS, NBS1, EB)

  featp = jnp.pad(features, ((0, NP0 - N0), (0, 0)))

  def cut(dg, size, npad):
    return dg[:npad, None], dg[size:size + npad, None]

  a0s, b0s = cut(dg0s, S0, NP0)
  a0d, b0d = cut(dg0d, S0, NP0)
  a1s, b1s = cut(dg1s, S1, NP1)
  a1d, b1d = cut(dg1d, S1, NP1)
  a2s, b2s = cut(dg2s, S2, NP2)
  a2d, b2d = cut(dg2d, S2, NP2)

  ns0, nd0 = _tc_invs(a0s, b0s, a0d, b0d, NP0, 1264)
  ns1, nd1 = _tc_invs(a1s, b1s, a1d, b1d, NP1, 640)
  ns2, nd2 = _tc_invs(a2s, b2s, a2d, b2d, NP2, 320)
  xlo, xhi = _tc_scale0(featp, ns0)
  xt0 = jnp.concatenate([xlo, xhi], axis=0)

  p0 = _prop0(xt0, s0c, d0c)
  z = _tc_layer0(p0[0, :N0], p0[1, :N0], nd0[:N0], W1[:64], W1[64:],
                 b1.reshape(1, 128))

  emb, x1lo, x1hi = _tc_proj0(z, P0, ns1[:N1])
  xt1a = jnp.concatenate([jnp.pad(x1lo, ((0, NP1 - N1), (0, 0))),
                          jnp.pad(x1hi, ((0, NP1 - N1), (0, 0)))], axis=0)

  part1 = _prop1(xt1a, s1c, d1c)
  snd = jnp.pad(ns1[:N1] * nd1[:N1], ((0, NP1 - N1), (0, 0)))
  snd2 = jnp.concatenate([snd, snd], axis=0)
  xt1b = _tc_mid(part1.reshape(2 * NP1, 64), snd2)

  part1b = _prop1(xt1b, s1c, d1c)
  x2s = _tc_proj1(part1b[0, :N1], part1b[1, :N1], nd1[:N1], P1, W2,
                  ns2[:N2])

  xt2 = jnp.pad(x2s, ((0, NP2 - N2), (0, 24)))
  part2 = _prop2(xt2, s2, d2)
  out = _tc_final(part2[0, :N2, :40], part2[1, :N2, :40], nd2[:N2],
                  b2.reshape(1, 40))
  return (out, emb)


# speedup vs baseline: 8.2279x; 1.0011x over previous
import jax
import jax.numpy as jnp
from jax import lax
from jax.experimental import pallas as pl
from jax.experimental.pallas import tpu as pltpu
from jax.experimental.pallas import tpu_sc as plsc

NC = 2
NS = 16
NW = NC * NS
EB = 128
NBUF = 4

N0, N1, N2 = 10000, 5000, 2500
NP0, NP1, NP2 = 10112, 5120, 2560
NB0, NB1, NB2 = 80, 40, 20
NBS0, NBS1 = 160, 80
S0, S1, S2 = 10240, 6144, 4096

_MESH = plsc.VectorSubcoreMesh(
    core_axis_name="c", subcore_axis_name="s", num_cores=NC, num_subcores=NS)


def _zero_vec(ref, n):
  @pl.loop(0, n // 16)
  def _(i):
    ref[pl.ds(i * 16, 16)] = jnp.zeros((16,), jnp.float32)


def _deg_body(e0s, e0d, e1s, e1d, e2s, e2d,
              o0s, o0d, o1s, o1d, o2s, o2d,
              idxv, onesv, zbuf,
              sec0s, sec0d, sec1s, sec1d, sec2s, sec2d, dsem):
  cid = lax.axis_index("c")
  sid = lax.axis_index("s")
  wid = sid * NC + cid

  _zero_vec(zbuf, 640)
  @pl.loop(0, EB // 16)
  def _(i):
    onesv[pl.ds(i * 16, 16)] = jnp.ones((16,), jnp.float32)

  passes = [(e0s, sec0s, o0s, S0, NB0), (e0d, sec0d, o0d, S0, NB0),
            (e1s, sec1s, o1s, S1, NB1), (e1d, sec1d, o1d, S1, NB1),
            (e2s, sec2s, o2s, S2, NB2), (e2d, sec2d, o2d, S2, NB2)]

  for _, sec, _, size, _ in passes:
    sz = size // NS
    pltpu.sync_copy(zbuf.at[pl.ds(0, sz)], sec.at[pl.ds(sid * sz, sz)])
  plsc.subcore_barrier()

  for e_ref, sec, _, _, nb in passes:
    pltpu.sync_copy(e_ref.at[wid], idxv.at[pl.ds(0, nb)])
    @pl.loop(0, nb // 4)
    def _(g):
      descs = []
      for b in range(4):
        descs.append(
            pltpu.async_copy(onesv, sec.at[idxv.at[g * 4 + b]], dsem,
                             add=True))
      for d_ in descs:
        d_.wait()
  plsc.subcore_barrier()

  for _, sec, out, size, _ in passes:
    sz = size // NS
    off = pl.multiple_of(cid * size + sid * sz, 128)
    pltpu.sync_copy(sec.at[pl.ds(sid * sz, sz)], out.at[pl.ds(off, sz)])


_deg_kernel = pl.kernel(
    _deg_body,
    out_type=[jax.ShapeDtypeStruct((NC * S0,), jnp.float32),
              jax.ShapeDtypeStruct((NC * S0,), jnp.float32),
              jax.ShapeDtypeStruct((NC * S1,), jnp.float32),
              jax.ShapeDtypeStruct((NC * S1,), jnp.float32),
              jax.ShapeDtypeStruct((NC * S2,), jnp.float32),
              jax.ShapeDtypeStruct((NC * S2,), jnp.float32)],
    mesh=_MESH,
    scratch_types=[
        pltpu.VMEM((NB0, EB), jnp.int32),
        pltpu.VMEM((EB,), jnp.float32),
        pltpu.VMEM((640,), jnp.float32),
        pltpu.VMEM_SHARED((S0,), jnp.float32),
        pltpu.VMEM_SHARED((S0,), jnp.float32),
        pltpu.VMEM_SHARED((S1,), jnp.float32),
        pltpu.VMEM_SHARED((S1,), jnp.float32),
        pltpu.VMEM_SHARED((S2,), jnp.float32),
        pltpu.VMEM_SHARED((S2,), jnp.float32),
        pltpu.SemaphoreType.DMA,
    ])


def _zero_acc_slice(rows0, acc, d, base, rows_per_tile):
  @pl.loop(0, EB)
  def _(r):
    @pl.loop(0, d // 16)
    def _(c):
      rows0[r, pl.ds(c * 16, 16)] = jnp.zeros((16,), jnp.float32)
  off = 0
  while off < rows_per_tile:
    c = min(EB, rows_per_tile - off)
    pltpu.sync_copy(rows0.at[pl.ds(0, c)], acc.at[pl.ds(base + off, c)])
    off += c


def _ring(ngroups, srcv, dstv, rows, gsem, ssem, gather_from, acc):
  def issue_gather(j, b):
    return pltpu.async_copy(gather_from.at[srcv.at[j]], rows[b], gsem[b])

  def issue_scatter(j, b):
    return pltpu.async_copy(rows[b], acc.at[dstv.at[j]], ssem[b], add=True)

  gds = [issue_gather(b, b) for b in range(NBUF)]

  @pl.loop(0, ngroups - 1)
  def _(g):
    sds = []
    for b in range(NBUF):
      gds[b].wait()
      sds.append(issue_scatter(g * NBUF + b, b))
    for b in range(NBUF):
      sds[b].wait()
      issue_gather((g + 1) * NBUF + b, b)

  last = ngroups - 1
  sds = []
  for b in range(NBUF):
    gds[b].wait()
    sds.append(issue_scatter(last * NBUF + b, b))
  for b in range(NBUF):
    sds[b].wait()


def _make_prop0():
  d = 32
  nbs = NBS0
  ngroups = nbs // NBUF
  rows_per_tile = NP0 // NS

  def body(xs_hbm, src_hbm, dst_hbm, out_hbm, srcv, dstv, *rest):
    rows = list(rest[0:NBUF])
    table = rest[NBUF]
    acc = rest[NBUF + 1]
    gsem = list(rest[NBUF + 2: NBUF + 2 + NBUF])
    ssem = list(rest[NBUF + 2 + NBUF:])
    cid = lax.axis_index("c")
    sid = lax.axis_index("s")

    pltpu.sync_copy(dst_hbm.at[sid], dstv)
    base = pl.multiple_of(sid * rows_per_tile, 8)

    for p in range(2):
      q = 2 * cid + p
      pltpu.sync_copy(src_hbm.at[q * NS + sid], srcv)
      hoff = pl.multiple_of(q * NP0 + sid * rows_per_tile, 8)
      pltpu.sync_copy(xs_hbm.at[pl.ds(hoff, rows_per_tile)],
                      table.at[pl.ds(base, rows_per_tile)])
      _zero_acc_slice(rows[0], acc, d, base, rows_per_tile)
      plsc.subcore_barrier()

      _ring(ngroups, srcv, dstv, rows, gsem, ssem, table, acc)
      plsc.subcore_barrier()

      off = 0
      while off < rows_per_tile:
        c = min(512, rows_per_tile - off)
        pltpu.sync_copy(acc.at[pl.ds(base + off, c)],
                        out_hbm.at[q, pl.ds(base + off, c)])
        off += c

  return pl.kernel(
      body,
      out_type=jax.ShapeDtypeStruct((4, NP0, d), jnp.float32),
      mesh=_MESH,
      compiler_params=pltpu.CompilerParams(use_tc_tiling_on_sc=False),
      scratch_types=(
          [pltpu.VMEM((nbs, EB), jnp.int32) for _ in range(2)]
          + [pltpu.VMEM((EB, d), jnp.float32) for _ in range(NBUF)]
          + [pltpu.VMEM_SHARED((NP0, d), jnp.float32)]
          + [pltpu.VMEM_SHARED((NP0, d), jnp.float32)]
          + [pltpu.SemaphoreType.DMA for _ in range(2 * NBUF)]))


def _make_prop0_r2():
  d = 64
  nbs = NBS0
  ngroups = nbs // NBUF
  rows_per_tile = NP0 // NS

  def body(xs_hbm, src_hbm, dst_hbm, out_hbm, srcv, dstv, *rest):
    rows = list(rest[0:NBUF])
    acc = rest[NBUF]
    gsem = list(rest[NBUF + 1: NBUF + 1 + NBUF])
    ssem = list(rest[NBUF + 1 + NBUF:])
    cid = lax.axis_index("c")
    sid = lax.axis_index("s")

    pltpu.sync_copy(src_hbm.at[cid * NS + sid], srcv)
    pltpu.sync_copy(dst_hbm.at[sid], dstv)

    base = pl.multiple_of(sid * rows_per_tile, 8)
    _zero_acc_slice(rows[0], acc, d, base, rows_per_tile)
    plsc.subcore_barrier()

    _ring(ngroups, srcv, dstv, rows, gsem, ssem, xs_hbm, acc)
    plsc.subcore_barrier()

    off = 0
    while off < rows_per_tile:
      c = min(512, rows_per_tile - off)
      pltpu.sync_copy(acc.at[pl.ds(base + off, c)],
                      out_hbm.at[cid, pl.ds(base + off, c)])
      off += c

  return pl.kernel(
      body,
      out_type=jax.ShapeDtypeStruct((NC, NP0, d), jnp.float32),
      mesh=_MESH,
      compiler_params=pltpu.CompilerParams(use_tc_tiling_on_sc=False),
      scratch_types=(
          [pltpu.VMEM((nbs, EB), jnp.int32) for _ in range(2)]
          + [pltpu.VMEM((EB, d), jnp.float32) for _ in range(NBUF)]
          + [pltpu.VMEM_SHARED((NP0, d), jnp.float32)]
          + [pltpu.SemaphoreType.DMA for _ in range(2 * NBUF)]))


def _make_prop1():
  d = 64
  nbs = NBS1
  ngroups = nbs // NBUF
  rows_per_tile = NP1 // NS

  def body(xs_hbm, snd_hbm, src_hbm, dst_hbm, out_hbm, srcv, dstv, *rest):
    rows = list(rest[0:NBUF])
    mbuf = rest[NBUF]
    sndv = rest[NBUF + 1]
    table = rest[NBUF + 2]
    acc = rest[NBUF + 3]
    gsem = list(rest[NBUF + 4: NBUF + 4 + NBUF])
    ssem = list(rest[NBUF + 4 + NBUF:])
    cid = lax.axis_index("c")
    sid = lax.axis_index("s")

    pltpu.sync_copy(src_hbm.at[sid], srcv)
    pltpu.sync_copy(dst_hbm.at[sid], dstv)

    base = pl.multiple_of(sid * rows_per_tile, 8)
    hoff = pl.multiple_of(cid * NP1 + sid * rows_per_tile, 8)
    pltpu.sync_copy(xs_hbm.at[pl.ds(hoff, rows_per_tile)],
                    table.at[pl.ds(base, rows_per_tile)])
    pltpu.sync_copy(snd_hbm.at[pl.ds(base, rows_per_tile)], sndv)
    _zero_acc_slice(rows[0], acc, d, base, rows_per_tile)
    plsc.subcore_barrier()

    _ring(ngroups, srcv, dstv, rows, gsem, ssem, table, acc)
    plsc.subcore_barrier()

    pltpu.sync_copy(acc.at[pl.ds(base, rows_per_tile)], mbuf)
    @pl.loop(0, rows_per_tile)
    def _(r):
      s = sndv[r, pl.ds(0, 16)]
      for c in range(d // 16):
        mbuf[r, pl.ds(c * 16, 16)] = mbuf[r, pl.ds(c * 16, 16)] * s
    pltpu.sync_copy(mbuf, table.at[pl.ds(base, rows_per_tile)])
    _zero_acc_slice(rows[0], acc, d, base, rows_per_tile)
    plsc.subcore_barrier()

    _ring(ngroups, srcv, dstv, rows, gsem, ssem, table, acc)
    plsc.subcore_barrier()

    off = 0
    while off < rows_per_tile:
      c = min(512, rows_per_tile - off)
      pltpu.sync_copy(acc.at[pl.ds(base + off, c)],
                      out_hbm.at[cid, pl.ds(base + off, c)])
      off += c

  return pl.kernel(
      body,
      out_type=jax.ShapeDtypeStruct((NC, NP1, d), jnp.float32),
      mesh=_MESH,
      compiler_params=pltpu.CompilerParams(use_tc_tiling_on_sc=False),
      scratch_types=(
          [pltpu.VMEM((nbs, EB), jnp.int32) for _ in range(2)]
          + [pltpu.VMEM((EB, d), jnp.float32) for _ in range(NBUF)]
          + [pltpu.VMEM((NP1 // NS, d), jnp.float32)]
          + [pltpu.VMEM((NP1 // NS, 16), jnp.float32)]
          + [pltpu.VMEM_SHARED((NP1, d), jnp.float32)]
          + [pltpu.VMEM_SHARED((NP1, d), jnp.float32)]
          + [pltpu.SemaphoreType.DMA for _ in range(2 * NBUF)]))


def _make_prop2():
  d = 64
  nb = NB2
  ngroups = nb // NBUF
  rows_per_tile = NP2 // NS

  def body(xs_hbm, src_hbm, dst_hbm, out_hbm, srcv, dstv, *rest):
    rows = list(rest[0:NBUF])
    table = rest[NBUF]
    acc = rest[NBUF + 1]
    gsem = list(rest[NBUF + 2: NBUF + 2 + NBUF])
    ssem = list(rest[NBUF + 2 + NBUF:])
    cid = lax.axis_index("c")
    sid = lax.axis_index("s")
    wid = sid * NC + cid

    pltpu.sync_copy(src_hbm.at[wid], srcv)
    pltpu.sync_copy(dst_hbm.at[wid], dstv)

    base = pl.multiple_of(sid * rows_per_tile, 8)
    pltpu.sync_copy(xs_hbm.at[pl.ds(base, rows_per_tile)],
                    table.at[pl.ds(base, rows_per_tile)])
    _zero_acc_slice(rows[0], acc, d, base, rows_per_tile)
    plsc.subcore_barrier()

    _ring(ngroups, srcv, dstv, rows, gsem, ssem, table, acc)
    plsc.subcore_barrier()

    off = 0
    while off < rows_per_tile:
      c = min(512, rows_per_tile - off)
      pltpu.sync_copy(acc.at[pl.ds(base + off, c)],
                      out_hbm.at[cid, pl.ds(base + off, c)])
      off += c

  return pl.kernel(
      body,
      out_type=jax.ShapeDtypeStruct((NC, NP2, d), jnp.float32),
      mesh=_MESH,
      compiler_params=pltpu.CompilerParams(use_tc_tiling_on_sc=False),
      scratch_types=(
          [pltpu.VMEM((nb, EB), jnp.int32) for _ in range(2)]
          + [pltpu.VMEM((EB, d), jnp.float32) for _ in range(NBUF)]
          + [pltpu.VMEM_SHARED((NP2, d), jnp.float32)]
          + [pltpu.VMEM_SHARED((NP2, d), jnp.float32)]
          + [pltpu.SemaphoreType.DMA for _ in range(2 * NBUF)]))


_prop0 = _make_prop0()
_prop0r2 = _make_prop0_r2()
_prop1 = _make_prop1()
_prop2 = _make_prop2()


def _tc_invs(a, b, c, d, n, blk):
  def body(a_r, b_r, c_r, d_r, ns_o, nd_o):
    def invs(x, y):
      dg = x[...] + y[...]
      return jnp.where(dg > 0, lax.rsqrt(dg), 0.0)
    ns_o[...] = invs(a_r, b_r)
    nd_o[...] = invs(c_r, d_r)

  spec = pl.BlockSpec((blk, 1), lambda i: (i, 0))
  return pl.pallas_call(
      body,
      grid=(n // blk,),
      in_specs=[spec, spec, spec, spec],
      out_specs=[spec, spec],
      out_shape=[jax.ShapeDtypeStruct((n, 1), jnp.float32),
                 jax.ShapeDtypeStruct((n, 1), jnp.float32)],
  )(a, b, c, d)


def _tc_scale0(featp, ns0):
  blk = 1264

  def body(f_r, ns_r, *outs):
    xs = f_r[...] * ns_r[...]
    for q in range(4):
      outs[q][...] = xs[:, q * 32:(q + 1) * 32]

  return pl.pallas_call(
      body,
      grid=(NP0 // blk,),
      in_specs=[pl.BlockSpec((blk, 128), lambda i: (i, 0)),
                pl.BlockSpec((blk, 1), lambda i: (i, 0))],
      out_specs=[pl.BlockSpec((blk, 32), lambda i: (i, 0))] * 4,
      out_shape=[jax.ShapeDtypeStruct((NP0, 32), jnp.float32)] * 4,
  )(featp, ns0)


def _tc_layer0(q0, q1, q2, q3, nd0, W1, b1r):
  def body(p0_r, p1_r, p2_r, p3_r, nd_r, w_r, b_r, z_r):
    nd = nd_r[...]
    acc = b_r[...] + jnp.zeros((1000, 128), jnp.float32)
    for q, p_r in enumerate((p0_r, p1_r, p2_r, p3_r)):
      acc += jnp.dot(p_r[...] * nd, w_r[q * 32:(q + 1) * 32, :],
                     preferred_element_type=jnp.float32)
    z_r[...] = jnp.maximum(acc, 0.0)

  qspec = pl.BlockSpec((1000, 32), lambda i: (i, 0))
  return pl.pallas_call(
      body,
      grid=(10,),
      in_specs=[qspec, qspec, qspec, qspec,
                pl.BlockSpec((1000, 1), lambda i: (i, 0)),
                pl.BlockSpec((128, 128), lambda i: (0, 0)),
                pl.BlockSpec((1, 128), lambda i: (0, 0))],
      out_specs=pl.BlockSpec((1000, 128), lambda i: (i, 0)),
      out_shape=jax.ShapeDtypeStruct((N0, 128), jnp.float32),
  )(q0, q1, q2, q3, nd0, W1, b1r)


def _tc_snd16(ns1, nd1):
  blk = 640

  def body(ns_r, nd_r, o_r):
    i = pl.program_id(0)
    row = i * blk + lax.broadcasted_iota(jnp.int32, (blk, 16), 0)
    s = ns_r[...] * nd_r[...]
    o_r[...] = jnp.where(row < N1, s, 0.0)

  return pl.pallas_call(
      body,
      grid=(NP1 // blk,),
      in_specs=[pl.BlockSpec((blk, 1), lambda i: (i, 0)),
                pl.BlockSpec((blk, 1), lambda i: (i, 0))],
      out_specs=pl.BlockSpec((blk, 16), lambda i: (i, 0)),
      out_shape=jax.ShapeDtypeStruct((NP1, 16), jnp.float32),
  )(ns1, nd1)


def _tc_proj0(z, P0, ns1):
  nk = 10

  def body(p_r, z_r, ns_r, emb_r, xlo_r, xhi_r):
    k = pl.program_id(1)
    part = lax.dot_general(p_r[...], z_r[...], (((0,), (0,)), ((), ())),
                           preferred_element_type=jnp.float32)
    @pl.when(k == 0)
    def _():
      emb_r[...] = part
    @pl.when(k > 0)
    def _():
      emb_r[...] += part
    @pl.when(k == nk - 1)
    def _():
      xs = emb_r[...] * ns_r[...]
      xlo_r[...] = xs[:, :64]
      xhi_r[...] = xs[:, 64:]

  return pl.pallas_call(
      body,
      grid=(10, nk),
      in_specs=[pl.BlockSpec((1000, 512), lambda j, k: (k, j)),
                pl.BlockSpec((1000, 128), lambda j, k: (k, 0)),
                pl.BlockSpec((512, 1), lambda j, k: (j, 0))],
      out_specs=[pl.BlockSpec((512, 128), lambda j, k: (j, 0)),
                 pl.BlockSpec((512, 64), lambda j, k: (j, 0)),
                 pl.BlockSpec((512, 64), lambda j, k: (j, 0))],
      out_shape=[jax.ShapeDtypeStruct((N1, 128), jnp.float32),
                 jax.ShapeDtypeStruct((N1, 64), jnp.float32),
                 jax.ShapeDtypeStruct((N1, 64), jnp.float32)],
  )(P0, z, ns1)


def _tc_proj1(plo, phi, nd1, P1, W2, ns2):
  nk = 5

  def body(p_r, plo_r, phi_r, nd_r, w2_r, ns_r, x2_r, acc_r):
    k = pl.program_id(1)
    h1 = jnp.concatenate([plo_r[...], phi_r[...]], axis=1) * nd_r[...]
    part = lax.dot_general(p_r[...], h1, (((0,), (0,)), ((), ())),
                           preferred_element_type=jnp.float32)
    @pl.when(k == 0)
    def _():
      acc_r[...] = part
    @pl.when(k > 0)
    def _():
      acc_r[...] += part
    @pl.when(k == nk - 1)
    def _():
      x2_r[...] = jnp.dot(acc_r[...], w2_r[...],
                          preferred_element_type=jnp.float32) * ns_r[...]

  return pl.pallas_call(
      body,
      grid=(5, nk),
      in_specs=[pl.BlockSpec((1000, 512), lambda j, k: (k, j)),
                pl.BlockSpec((1000, 64), lambda j, k: (k, 0)),
                pl.BlockSpec((1000, 64), lambda j, k: (k, 0)),
                pl.BlockSpec((1000, 1), lambda j, k: (k, 0)),
                pl.BlockSpec((128, 40), lambda j, k: (0, 0)),
                pl.BlockSpec((512, 1), lambda j, k: (j, 0))],
      out_specs=pl.BlockSpec((512, 40), lambda j, k: (j, 0)),
      out_shape=jax.ShapeDtypeStruct((N2, 40), jnp.float32),
      scratch_shapes=[pltpu.VMEM((512, 128), jnp.float32)],
  )(P1, plo, phi, nd1, W2, ns2)


def _tc_final(pa, pb, nd2, b2r):
  def body(pa_r, pb_r, nd_r, b_r, o_r):
    o_r[...] = (pa_r[...] + pb_r[...]) * nd_r[...] + b_r[...]

  return pl.pallas_call(
      body,
      out_shape=jax.ShapeDtypeStruct((N2, 40), jnp.float32),
  )(pa, pb, nd2, b2r)


def _prep_edges(ei, nb, dummy):
  e_pad = NW * nb * EB
  pad = e_pad - ei.shape[1]
  fill = jnp.full((pad,), dummy, dtype=jnp.int32)
  src = jnp.concatenate([ei[0], fill]).reshape(NW, nb, EB)
  dst = jnp.concatenate([ei[1], fill]).reshape(NW, nb, EB)
  return src, dst


def kernel(features, edge_index0, edge_index1, edge_index2, P0, P1, W1, b1,
           W2, b2):
  s0, d0 = _prep_edges(edge_index0, NB0, N0)
  s1, d1 = _prep_edges(edge_index1, NB1, N1)
  s2, d2 = _prep_edges(edge_index2, NB2, N2)

  dg0s, dg0d, dg1s, dg1d, dg2s, dg2d = _deg_kernel(s0, d0, s1, d1, s2, d2)

  s0s = s0.reshape(NS, NBS0, EB)
  s0c = jnp.concatenate([s0s + q * NP0 for q in range(4)], axis=0)
  d0c = d0.reshape(NS, NBS0, EB)
  s1c = s1.reshape(NS, NBS1, EB)
  d1c = d1.reshape(NS, NBS1, EB)

  featp = jnp.pad(features, ((0, NP0 - N0), (0, 0)))

  def cut(dg, size, npad):
    return dg[:npad, None], dg[size:size + npad, None]

  a0s, b0s = cut(dg0s, S0, NP0)
  a0d, b0d = cut(dg0d, S0, NP0)
  a1s, b1s = cut(dg1s, S1, NP1)
  a1d, b1d = cut(dg1d, S1, NP1)
  a2s, b2s = cut(dg2s, S2, NP2)
  a2d, b2d = cut(dg2d, S2, NP2)

  ns0, nd0 = _tc_invs(a0s, b0s, a0d, b0d, NP0, 1264)
  ns1, nd1 = _tc_invs(a1s, b1s, a1d, b1d, NP1, 640)
  ns2, nd2 = _tc_invs(a2s, b2s, a2d, b2d, NP2, 320)
  quarters = _tc_scale0(featp, ns0)
  xt0h = jnp.concatenate(
      [jnp.concatenate([quarters[0], quarters[1]], axis=1),
       jnp.concatenate([quarters[2], quarters[3]], axis=1)], axis=0)
  s0c2 = jnp.concatenate([s0s, s0s + NP0], axis=0)

  p0h = _prop0r2(xt0h, s0c2, d0c)
  z = _tc_layer0(p0h[0, :N0, :32], p0h[0, :N0, 32:],
                 p0h[1, :N0, :32], p0h[1, :N0, 32:],
                 nd0[:N0], W1, b1.reshape(1, 128))

  emb, x1lo, x1hi = _tc_proj0(z, P0, ns1[:N1])
  xt1a = jnp.concatenate([jnp.pad(x1lo, ((0, NP1 - N1), (0, 0))),
                          jnp.pad(x1hi, ((0, NP1 - N1), (0, 0)))], axis=0)
  snd16 = _tc_snd16(ns1, nd1)

  part1b = _prop1(xt1a, snd16, s1c, d1c)
  x2s = _tc_proj1(part1b[0, :N1], part1b[1, :N1], nd1[:N1], P1, W2,
                  ns2[:N2])

  xt2 = jnp.pad(x2s, ((0, NP2 - N2), (0, 24)))
  part2 = _prop2(xt2, s2, d2)
  out = _tc_final(part2[0, :N2, :40], part2[1, :N2, :40], nd2[:N2],
                  b2.reshape(1, 40))
  return (out, emb)


# speedup vs baseline: 9.6135x; 1.1684x over previous
import jax
import jax.numpy as jnp
from jax import lax
from jax.experimental import pallas as pl
from jax.experimental.pallas import tpu as pltpu
from jax.experimental.pallas import tpu_sc as plsc

NC = 2
NS = 16
NW = NC * NS
EB = 128
NBUF = 4

N0, N1, N2 = 10000, 5000, 2500
NP0, NP1, NP2 = 10112, 5120, 2560
NB0, NB1, NB2 = 80, 40, 20
NBS0, NBS1 = 160, 80
S0, S1, S2 = 10240, 6144, 4096

_MESH = plsc.VectorSubcoreMesh(
    core_axis_name="c", subcore_axis_name="s", num_cores=NC, num_subcores=NS)


def _zero_vec(ref, n):
  @pl.loop(0, n // 16)
  def _(i):
    ref[pl.ds(i * 16, 16)] = jnp.zeros((16,), jnp.float32)


def _deg_body(e0s, e0d, e1s, e1d, e2s, e2d,
              o0s, o0d, o1s, o1d, o2s, o2d,
              idxv, onesv, zbuf,
              sec0s, sec0d, sec1s, sec1d, sec2s, sec2d, dsem):
  cid = lax.axis_index("c")
  sid = lax.axis_index("s")
  wid = sid * NC + cid

  _zero_vec(zbuf, 640)
  @pl.loop(0, EB // 16)
  def _(i):
    onesv[pl.ds(i * 16, 16)] = jnp.ones((16,), jnp.float32)

  passes = [(e0s, sec0s, o0s, S0, NB0), (e0d, sec0d, o0d, S0, NB0),
            (e1s, sec1s, o1s, S1, NB1), (e1d, sec1d, o1d, S1, NB1),
            (e2s, sec2s, o2s, S2, NB2), (e2d, sec2d, o2d, S2, NB2)]

  for _, sec, _, size, _ in passes:
    sz = size // NS
    pltpu.sync_copy(zbuf.at[pl.ds(0, sz)], sec.at[pl.ds(sid * sz, sz)])
  plsc.subcore_barrier()

  for e_ref, sec, _, _, nb in passes:
    pltpu.sync_copy(e_ref.at[wid], idxv.at[pl.ds(0, nb)])
    @pl.loop(0, nb // 4)
    def _(g):
      descs = []
      for b in range(4):
        descs.append(
            pltpu.async_copy(onesv, sec.at[idxv.at[g * 4 + b]], dsem,
                             add=True))
      for d_ in descs:
        d_.wait()
  plsc.subcore_barrier()

  for _, sec, out, size, _ in passes:
    sz = size // NS
    off = pl.multiple_of(cid * size + sid * sz, 128)
    pltpu.sync_copy(sec.at[pl.ds(sid * sz, sz)], out.at[pl.ds(off, sz)])


_deg_kernel = pl.kernel(
    _deg_body,
    out_type=[jax.ShapeDtypeStruct((NC * S0,), jnp.float32),
              jax.ShapeDtypeStruct((NC * S0,), jnp.float32),
              jax.ShapeDtypeStruct((NC * S1,), jnp.float32),
              jax.ShapeDtypeStruct((NC * S1,), jnp.float32),
              jax.ShapeDtypeStruct((NC * S2,), jnp.float32),
              jax.ShapeDtypeStruct((NC * S2,), jnp.float32)],
    mesh=_MESH,
    scratch_types=[
        pltpu.VMEM((NB0, EB), jnp.int32),
        pltpu.VMEM((EB,), jnp.float32),
        pltpu.VMEM((640,), jnp.float32),
        pltpu.VMEM_SHARED((S0,), jnp.float32),
        pltpu.VMEM_SHARED((S0,), jnp.float32),
        pltpu.VMEM_SHARED((S1,), jnp.float32),
        pltpu.VMEM_SHARED((S1,), jnp.float32),
        pltpu.VMEM_SHARED((S2,), jnp.float32),
        pltpu.VMEM_SHARED((S2,), jnp.float32),
        pltpu.SemaphoreType.DMA,
    ])


def _zero_acc_slice(rows0, acc, d, base, rows_per_tile):
  @pl.loop(0, EB)
  def _(r):
    @pl.loop(0, d // 16)
    def _(c):
      rows0[r, pl.ds(c * 16, 16)] = jnp.zeros((16,), jnp.float32)
  off = 0
  while off < rows_per_tile:
    c = min(EB, rows_per_tile - off)
    pltpu.sync_copy(rows0.at[pl.ds(0, c)], acc.at[pl.ds(base + off, c)])
    off += c


def _ring(ngroups, srcv, dstv, rows, gsem, ssem, gather_from, acc):
  def issue_gather(j, b):
    return pltpu.async_copy(gather_from.at[srcv.at[j]], rows[b], gsem[b])

  def issue_scatter(j, b):
    return pltpu.async_copy(rows[b], acc.at[dstv.at[j]], ssem[b], add=True)

  gds = [issue_gather(b, b) for b in range(NBUF)]

  @pl.loop(0, ngroups - 1)
  def _(g):
    sds = []
    for b in range(NBUF):
      gds[b].wait()
      sds.append(issue_scatter(g * NBUF + b, b))
    for b in range(NBUF):
      sds[b].wait()
      issue_gather((g + 1) * NBUF + b, b)

  last = ngroups - 1
  sds = []
  for b in range(NBUF):
    gds[b].wait()
    sds.append(issue_scatter(last * NBUF + b, b))
  for b in range(NBUF):
    sds[b].wait()


def _make_prop0():
  d = 32
  nbs = NBS0
  ngroups = nbs // NBUF
  rows_per_tile = NP0 // NS

  def body(xs_hbm, src_hbm, dst_hbm, out_hbm, srcv, dstv, *rest):
    rows = list(rest[0:NBUF])
    table = rest[NBUF]
    acc = rest[NBUF + 1]
    gsem = list(rest[NBUF + 2: NBUF + 2 + NBUF])
    ssem = list(rest[NBUF + 2 + NBUF:])
    cid = lax.axis_index("c")
    sid = lax.axis_index("s")

    pltpu.sync_copy(dst_hbm.at[sid], dstv)
    base = pl.multiple_of(sid * rows_per_tile, 8)

    for p in range(2):
      q = 2 * cid + p
      pltpu.sync_copy(src_hbm.at[q * NS + sid], srcv)
      hoff = pl.multiple_of(q * NP0 + sid * rows_per_tile, 8)
      pltpu.sync_copy(xs_hbm.at[pl.ds(hoff, rows_per_tile)],
                      table.at[pl.ds(base, rows_per_tile)])
      _zero_acc_slice(rows[0], acc, d, base, rows_per_tile)
      plsc.subcore_barrier()

      _ring(ngroups, srcv, dstv, rows, gsem, ssem, table, acc)
      plsc.subcore_barrier()

      off = 0
      while off < rows_per_tile:
        c = min(512, rows_per_tile - off)
        pltpu.sync_copy(acc.at[pl.ds(base + off, c)],
                        out_hbm.at[q, pl.ds(base + off, c)])
        off += c

  return pl.kernel(
      body,
      out_type=jax.ShapeDtypeStruct((4, NP0, d), jnp.float32),
      mesh=_MESH,
      compiler_params=pltpu.CompilerParams(use_tc_tiling_on_sc=False),
      scratch_types=(
          [pltpu.VMEM((nbs, EB), jnp.int32) for _ in range(2)]
          + [pltpu.VMEM((EB, d), jnp.float32) for _ in range(NBUF)]
          + [pltpu.VMEM_SHARED((NP0, d), jnp.float32)]
          + [pltpu.VMEM_SHARED((NP0, d), jnp.float32)]
          + [pltpu.SemaphoreType.DMA for _ in range(2 * NBUF)]))


def _make_prop0_r2():
  d = 64
  nbs = NBS0
  ngroups = nbs // NBUF
  rows_per_tile = NP0 // NS

  def body(xs_hbm, src_hbm, dst_hbm, out_hbm, srcv, dstv, *rest):
    rows = list(rest[0:NBUF])
    acc = rest[NBUF]
    gsem = list(rest[NBUF + 1: NBUF + 1 + NBUF])
    ssem = list(rest[NBUF + 1 + NBUF:])
    cid = lax.axis_index("c")
    sid = lax.axis_index("s")

    pltpu.sync_copy(src_hbm.at[cid * NS + sid], srcv)
    pltpu.sync_copy(dst_hbm.at[sid], dstv)

    base = pl.multiple_of(sid * rows_per_tile, 8)
    _zero_acc_slice(rows[0], acc, d, base, rows_per_tile)
    plsc.subcore_barrier()

    _ring(ngroups, srcv, dstv, rows, gsem, ssem, xs_hbm, acc)
    plsc.subcore_barrier()

    off = 0
    while off < rows_per_tile:
      c = min(512, rows_per_tile - off)
      pltpu.sync_copy(acc.at[pl.ds(base + off, c)],
                      out_hbm.at[cid, pl.ds(base + off, c)])
      off += c

  return pl.kernel(
      body,
      out_type=jax.ShapeDtypeStruct((NC, NP0, d), jnp.float32),
      mesh=_MESH,
      compiler_params=pltpu.CompilerParams(use_tc_tiling_on_sc=False),
      scratch_types=(
          [pltpu.VMEM((nbs, EB), jnp.int32) for _ in range(2)]
          + [pltpu.VMEM((EB, d), jnp.float32) for _ in range(NBUF)]
          + [pltpu.VMEM_SHARED((NP0, d), jnp.float32)]
          + [pltpu.SemaphoreType.DMA for _ in range(2 * NBUF)]))


def _make_prop1():
  d = 64
  nbs = NBS1
  ngroups = nbs // NBUF
  rows_per_tile = NP1 // NS

  def body(xs_hbm, snd_hbm, src_hbm, dst_hbm, out_hbm, srcv, dstv, *rest):
    rows = list(rest[0:NBUF])
    mbuf = rest[NBUF]
    sndv = rest[NBUF + 1]
    table = rest[NBUF + 2]
    acc = rest[NBUF + 3]
    gsem = list(rest[NBUF + 4: NBUF + 4 + NBUF])
    ssem = list(rest[NBUF + 4 + NBUF:])
    cid = lax.axis_index("c")
    sid = lax.axis_index("s")

    pltpu.sync_copy(src_hbm.at[sid], srcv)
    pltpu.sync_copy(dst_hbm.at[sid], dstv)

    base = pl.multiple_of(sid * rows_per_tile, 8)
    hoff = pl.multiple_of(cid * NP1 + sid * rows_per_tile, 8)
    pltpu.sync_copy(xs_hbm.at[pl.ds(hoff, rows_per_tile)],
                    table.at[pl.ds(base, rows_per_tile)])
    pltpu.sync_copy(snd_hbm.at[pl.ds(base, rows_per_tile)], sndv)
    _zero_acc_slice(rows[0], acc, d, base, rows_per_tile)
    plsc.subcore_barrier()

    _ring(ngroups, srcv, dstv, rows, gsem, ssem, table, acc)
    plsc.subcore_barrier()

    pltpu.sync_copy(acc.at[pl.ds(base, rows_per_tile)], mbuf)
    @pl.loop(0, rows_per_tile)
    def _(r):
      s = sndv[r, pl.ds(0, 16)]
      for c in range(d // 16):
        mbuf[r, pl.ds(c * 16, 16)] = mbuf[r, pl.ds(c * 16, 16)] * s
    pltpu.sync_copy(mbuf, table.at[pl.ds(base, rows_per_tile)])
    _zero_acc_slice(rows[0], acc, d, base, rows_per_tile)
    plsc.subcore_barrier()

    _ring(ngroups, srcv, dstv, rows, gsem, ssem, table, acc)
    plsc.subcore_barrier()

    off = 0
    while off < rows_per_tile:
      c = min(512, rows_per_tile - off)
      pltpu.sync_copy(acc.at[pl.ds(base + off, c)],
                      out_hbm.at[cid, pl.ds(base + off, c)])
      off += c

  return pl.kernel(
      body,
      out_type=jax.ShapeDtypeStruct((NC, NP1, d), jnp.float32),
      mesh=_MESH,
      compiler_params=pltpu.CompilerParams(use_tc_tiling_on_sc=False),
      scratch_types=(
          [pltpu.VMEM((nbs, EB), jnp.int32) for _ in range(2)]
          + [pltpu.VMEM((EB, d), jnp.float32) for _ in range(NBUF)]
          + [pltpu.VMEM((NP1 // NS, d), jnp.float32)]
          + [pltpu.VMEM((NP1 // NS, 16), jnp.float32)]
          + [pltpu.VMEM_SHARED((NP1, d), jnp.float32)]
          + [pltpu.VMEM_SHARED((NP1, d), jnp.float32)]
          + [pltpu.SemaphoreType.DMA for _ in range(2 * NBUF)]))


def _make_prop2():
  d = 64
  nb = NB2
  ngroups = nb // NBUF
  rows_per_tile = NP2 // NS

  def body(xs_hbm, src_hbm, dst_hbm, out_hbm, srcv, dstv, *rest):
    rows = list(rest[0:NBUF])
    table = rest[NBUF]
    acc = rest[NBUF + 1]
    gsem = list(rest[NBUF + 2: NBUF + 2 + NBUF])
    ssem = list(rest[NBUF + 2 + NBUF:])
    cid = lax.axis_index("c")
    sid = lax.axis_index("s")
    wid = sid * NC + cid

    pltpu.sync_copy(src_hbm.at[wid], srcv)
    pltpu.sync_copy(dst_hbm.at[wid], dstv)

    base = pl.multiple_of(sid * rows_per_tile, 8)
    pltpu.sync_copy(xs_hbm.at[pl.ds(base, rows_per_tile)],
                    table.at[pl.ds(base, rows_per_tile)])
    _zero_acc_slice(rows[0], acc, d, base, rows_per_tile)
    plsc.subcore_barrier()

    _ring(ngroups, srcv, dstv, rows, gsem, ssem, table, acc)
    plsc.subcore_barrier()

    off = 0
    while off < rows_per_tile:
      c = min(512, rows_per_tile - off)
      pltpu.sync_copy(acc.at[pl.ds(base + off, c)],
                      out_hbm.at[cid, pl.ds(base + off, c)])
      off += c

  return pl.kernel(
      body,
      out_type=jax.ShapeDtypeStruct((NC, NP2, d), jnp.float32),
      mesh=_MESH,
      compiler_params=pltpu.CompilerParams(use_tc_tiling_on_sc=False),
      scratch_types=(
          [pltpu.VMEM((nb, EB), jnp.int32) for _ in range(2)]
          + [pltpu.VMEM((EB, d), jnp.float32) for _ in range(NBUF)]
          + [pltpu.VMEM_SHARED((NP2, d), jnp.float32)]
          + [pltpu.VMEM_SHARED((NP2, d), jnp.float32)]
          + [pltpu.SemaphoreType.DMA for _ in range(2 * NBUF)]))


_prop0 = _make_prop0()
_prop0r2 = _make_prop0_r2()
_prop1 = _make_prop1()
_prop2 = _make_prop2()


def _tc_invs(a, b, c, d, n, blk):
  def body(a_r, b_r, c_r, d_r, ns_o, nd_o):
    def invs(x, y):
      dg = x[...] + y[...]
      return jnp.where(dg > 0, lax.rsqrt(dg), 0.0)
    ns_o[...] = invs(a_r, b_r)
    nd_o[...] = invs(c_r, d_r)

  spec = pl.BlockSpec((blk, 1), lambda i: (i, 0))
  return pl.pallas_call(
      body,
      grid=(n // blk,),
      in_specs=[spec, spec, spec, spec],
      out_specs=[spec, spec],
      out_shape=[jax.ShapeDtypeStruct((n, 1), jnp.float32),
                 jax.ShapeDtypeStruct((n, 1), jnp.float32)],
  )(a, b, c, d)


def _tc_scale0(featp, ns0):
  blk = 1264

  def body(f_r, ns_r, *outs):
    xs = f_r[...] * ns_r[...]
    for q in range(4):
      outs[q][...] = xs[:, q * 32:(q + 1) * 32]

  return pl.pallas_call(
      body,
      grid=(NP0 // blk,),
      in_specs=[pl.BlockSpec((blk, 128), lambda i: (i, 0)),
                pl.BlockSpec((blk, 1), lambda i: (i, 0))],
      out_specs=[pl.BlockSpec((blk, 32), lambda i: (i, 0))] * 4,
      out_shape=[jax.ShapeDtypeStruct((NP0, 32), jnp.float32)] * 4,
  )(featp, ns0)


def _tc_layer0(q0, q1, q2, q3, nd0, W1, b1r):
  def body(p0_r, p1_r, p2_r, p3_r, nd_r, w_r, b_r, z_r):
    nd = nd_r[...]
    acc = b_r[...] + jnp.zeros((1000, 128), jnp.float32)
    for q, p_r in enumerate((p0_r, p1_r, p2_r, p3_r)):
      acc += jnp.dot(p_r[...] * nd, w_r[q * 32:(q + 1) * 32, :],
                     preferred_element_type=jnp.float32)
    z_r[...] = jnp.maximum(acc, 0.0)

  qspec = pl.BlockSpec((1000, 32), lambda i: (i, 0))
  return pl.pallas_call(
      body,
      grid=(10,),
      in_specs=[qspec, qspec, qspec, qspec,
                pl.BlockSpec((1000, 1), lambda i: (i, 0)),
                pl.BlockSpec((128, 128), lambda i: (0, 0)),
                pl.BlockSpec((1, 128), lambda i: (0, 0))],
      out_specs=pl.BlockSpec((1000, 128), lambda i: (i, 0)),
      out_shape=jax.ShapeDtypeStruct((N0, 128), jnp.float32),
  )(q0, q1, q2, q3, nd0, W1, b1r)


def _tc_snd16(ns1, nd1):
  blk = 640

  def body(ns_r, nd_r, o_r):
    i = pl.program_id(0)
    row = i * blk + lax.broadcasted_iota(jnp.int32, (blk, 16), 0)
    s = ns_r[...] * nd_r[...]
    o_r[...] = jnp.where(row < N1, s, 0.0)

  return pl.pallas_call(
      body,
      grid=(NP1 // blk,),
      in_specs=[pl.BlockSpec((blk, 1), lambda i: (i, 0)),
                pl.BlockSpec((blk, 1), lambda i: (i, 0))],
      out_specs=pl.BlockSpec((blk, 16), lambda i: (i, 0)),
      out_shape=jax.ShapeDtypeStruct((NP1, 16), jnp.float32),
  )(ns1, nd1)


def _tc_proj0(z, P0T, ns1):
  blk = 256

  def body(p_r, z_r, ns_r, emb_r, xlo_r, xhi_r):
    part = lax.dot_general(p_r[...], z_r[...], (((1,), (0,)), ((), ())),
                           preferred_element_type=jnp.float32)
    emb_r[...] = part
    xs = part * ns_r[...]
    xlo_r[...] = xs[:, :64]
    xhi_r[...] = xs[:, 64:]

  return pl.pallas_call(
      body,
      grid=(20,),
      in_specs=[pl.BlockSpec((blk, N0), lambda j: (j, 0)),
                pl.BlockSpec((N0, 128), lambda j: (0, 0)),
                pl.BlockSpec((blk, 1), lambda j: (j, 0))],
      out_specs=[pl.BlockSpec((blk, 128), lambda j: (j, 0)),
                 pl.BlockSpec((blk, 64), lambda j: (j, 0)),
                 pl.BlockSpec((blk, 64), lambda j: (j, 0))],
      out_shape=[jax.ShapeDtypeStruct((N1, 128), jnp.float32),
                 jax.ShapeDtypeStruct((N1, 64), jnp.float32),
                 jax.ShapeDtypeStruct((N1, 64), jnp.float32)],
  )(P0T, z, ns1)


def _tc_proj1(plo, phi, nd1, P1, W2, ns2):
  nk = 5

  def body(p_r, plo_r, phi_r, nd_r, w2_r, ns_r, x2_r, acc_r):
    k = pl.program_id(1)
    h1 = jnp.concatenate([plo_r[...], phi_r[...]], axis=1) * nd_r[...]
    part = lax.dot_general(p_r[...], h1, (((0,), (0,)), ((), ())),
                           preferred_element_type=jnp.float32)
    @pl.when(k == 0)
    def _():
      acc_r[...] = part
    @pl.when(k > 0)
    def _():
      acc_r[...] += part
    @pl.when(k == nk - 1)
    def _():
      x2_r[...] = jnp.dot(acc_r[...], w2_r[...],
                          preferred_element_type=jnp.float32) * ns_r[...]

  return pl.pallas_call(
      body,
      grid=(5, nk),
      in_specs=[pl.BlockSpec((1000, 512), lambda j, k: (k, j)),
                pl.BlockSpec((1000, 64), lambda j, k: (k, 0)),
                pl.BlockSpec((1000, 64), lambda j, k: (k, 0)),
                pl.BlockSpec((1000, 1), lambda j, k: (k, 0)),
                pl.BlockSpec((128, 40), lambda j, k: (0, 0)),
                pl.BlockSpec((512, 1), lambda j, k: (j, 0))],
      out_specs=pl.BlockSpec((512, 40), lambda j, k: (j, 0)),
      out_shape=jax.ShapeDtypeStruct((N2, 40), jnp.float32),
      scratch_shapes=[pltpu.VMEM((512, 128), jnp.float32)],
  )(P1, plo, phi, nd1, W2, ns2)


def _tc_final(pa, pb, nd2, b2r):
  def body(pa_r, pb_r, nd_r, b_r, o_r):
    o_r[...] = (pa_r[...] + pb_r[...]) * nd_r[...] + b_r[...]

  return pl.pallas_call(
      body,
      out_shape=jax.ShapeDtypeStruct((N2, 40), jnp.float32),
  )(pa, pb, nd2, b2r)


def _prep_edges(ei, nb, dummy):
  e_pad = NW * nb * EB
  pad = e_pad - ei.shape[1]
  fill = jnp.full((pad,), dummy, dtype=jnp.int32)
  src = jnp.concatenate([ei[0], fill]).reshape(NW, nb, EB)
  dst = jnp.concatenate([ei[1], fill]).reshape(NW, nb, EB)
  return src, dst


def kernel(features, edge_index0, edge_index1, edge_index2, P0, P1, W1, b1,
           W2, b2):
  s0, d0 = _prep_edges(edge_index0, NB0, N0)
  s1, d1 = _prep_edges(edge_index1, NB1, N1)
  s2, d2 = _prep_edges(edge_index2, NB2, N2)

  dg0s, dg0d, dg1s, dg1d, dg2s, dg2d = _deg_kernel(s0, d0, s1, d1, s2, d2)

  s0s = s0.reshape(NS, NBS0, EB)
  s0c = jnp.concatenate([s0s + q * NP0 for q in range(4)], axis=0)
  d0c = d0.reshape(NS, NBS0, EB)
  s1c = s1.reshape(NS, NBS1, EB)
  d1c = d1.reshape(NS, NBS1, EB)

  featp = jnp.pad(features, ((0, NP0 - N0), (0, 0)))

  def cut(dg, size, npad):
    return dg[:npad, None], dg[size:size + npad, None]

  a0s, b0s = cut(dg0s, S0, NP0)
  a0d, b0d = cut(dg0d, S0, NP0)
  a1s, b1s = cut(dg1s, S1, NP1)
  a1d, b1d = cut(dg1d, S1, NP1)
  a2s, b2s = cut(dg2s, S2, NP2)
  a2d, b2d = cut(dg2d, S2, NP2)

  ns0, nd0 = _tc_invs(a0s, b0s, a0d, b0d, NP0, 1264)
  ns1, nd1 = _tc_invs(a1s, b1s, a1d, b1d, NP1, 640)
  ns2, nd2 = _tc_invs(a2s, b2s, a2d, b2d, NP2, 320)
  quarters = _tc_scale0(featp, ns0)
  xt0h = jnp.concatenate(
      [jnp.concatenate([quarters[0], quarters[1]], axis=1),
       jnp.concatenate([quarters[2], quarters[3]], axis=1)], axis=0)
  s0c2 = jnp.concatenate([s0s, s0s + NP0], axis=0)

  p0h = _prop0r2(xt0h, s0c2, d0c)
  z = _tc_layer0(p0h[0, :N0, :32], p0h[0, :N0, 32:],
                 p0h[1, :N0, :32], p0h[1, :N0, 32:],
                 nd0[:N0], W1, b1.reshape(1, 128))

  emb, x1lo, x1hi = _tc_proj0(z, P0.T, ns1[:N1])
  xt1a = jnp.concatenate([jnp.pad(x1lo, ((0, NP1 - N1), (0, 0))),
                          jnp.pad(x1hi, ((0, NP1 - N1), (0, 0)))], axis=0)
  snd16 = _tc_snd16(ns1, nd1)

  part1b = _prop1(xt1a, snd16, s1c, d1c)
  x2s = _tc_proj1(part1b[0, :N1], part1b[1, :N1], nd1[:N1], P1, W2,
                  ns2[:N2])

  xt2 = jnp.pad(x2s, ((0, NP2 - N2), (0, 24)))
  part2 = _prop2(xt2, s2, d2)
  out = _tc_final(part2[0, :N2, :40], part2[1, :N2, :40], nd2[:N2],
                  b2.reshape(1, 40))
  return (out, emb)


# speedup vs baseline: 9.9643x; 1.0365x over previous
import jax
import jax.numpy as jnp
from jax import lax
from jax.experimental import pallas as pl
from jax.experimental.pallas import tpu as pltpu
from jax.experimental.pallas import tpu_sc as plsc

NC = 2
NS = 16
NW = NC * NS
EB = 128
NBUF = 4

N0, N1, N2 = 10000, 5000, 2500
NP0, NP1, NP2 = 10112, 5120, 2560
NB0, NB1, NB2 = 80, 40, 20
NBS0, NBS1 = 160, 80
S0, S1, S2 = 10240, 6144, 4096

_MESH = plsc.VectorSubcoreMesh(
    core_axis_name="c", subcore_axis_name="s", num_cores=NC, num_subcores=NS)


def _zero_vec(ref, n):
  @pl.loop(0, n // 16)
  def _(i):
    ref[pl.ds(i * 16, 16)] = jnp.zeros((16,), jnp.float32)


def _deg_body(e0s, e0d, e1s, e1d, e2s, e2d,
              o0s, o0d, o1s, o1d, o2s, o2d,
              idxv, onesv, zbuf,
              sec0s, sec0d, sec1s, sec1d, sec2s, sec2d, dsem):
  cid = lax.axis_index("c")
  sid = lax.axis_index("s")
  wid = sid * NC + cid

  _zero_vec(zbuf, 640)
  @pl.loop(0, EB // 16)
  def _(i):
    onesv[pl.ds(i * 16, 16)] = jnp.ones((16,), jnp.float32)

  passes = [(e0s, sec0s, o0s, S0, NB0), (e0d, sec0d, o0d, S0, NB0),
            (e1s, sec1s, o1s, S1, NB1), (e1d, sec1d, o1d, S1, NB1),
            (e2s, sec2s, o2s, S2, NB2), (e2d, sec2d, o2d, S2, NB2)]

  for _, sec, _, size, _ in passes:
    sz = size // NS
    pltpu.sync_copy(zbuf.at[pl.ds(0, sz)], sec.at[pl.ds(sid * sz, sz)])
  plsc.subcore_barrier()

  for e_ref, sec, _, _, nb in passes:
    pltpu.sync_copy(e_ref.at[wid], idxv.at[pl.ds(0, nb)])
    @pl.loop(0, nb // 4)
    def _(g):
      descs = []
      for b in range(4):
        descs.append(
            pltpu.async_copy(onesv, sec.at[idxv.at[g * 4 + b]], dsem,
                             add=True))
      for d_ in descs:
        d_.wait()
  plsc.subcore_barrier()

  for _, sec, out, size, _ in passes:
    sz = size // NS
    off = pl.multiple_of(cid * size + sid * sz, 128)
    pltpu.sync_copy(sec.at[pl.ds(sid * sz, sz)], out.at[pl.ds(off, sz)])


_deg_kernel = pl.kernel(
    _deg_body,
    out_type=[jax.ShapeDtypeStruct((NC * S0,), jnp.float32),
              jax.ShapeDtypeStruct((NC * S0,), jnp.float32),
              jax.ShapeDtypeStruct((NC * S1,), jnp.float32),
              jax.ShapeDtypeStruct((NC * S1,), jnp.float32),
              jax.ShapeDtypeStruct((NC * S2,), jnp.float32),
              jax.ShapeDtypeStruct((NC * S2,), jnp.float32)],
    mesh=_MESH,
    scratch_types=[
        pltpu.VMEM((NB0, EB), jnp.int32),
        pltpu.VMEM((EB,), jnp.float32),
        pltpu.VMEM((640,), jnp.float32),
        pltpu.VMEM_SHARED((S0,), jnp.float32),
        pltpu.VMEM_SHARED((S0,), jnp.float32),
        pltpu.VMEM_SHARED((S1,), jnp.float32),
        pltpu.VMEM_SHARED((S1,), jnp.float32),
        pltpu.VMEM_SHARED((S2,), jnp.float32),
        pltpu.VMEM_SHARED((S2,), jnp.float32),
        pltpu.SemaphoreType.DMA,
    ])


def _zero_acc_slice(rows0, acc, d, base, rows_per_tile):
  @pl.loop(0, EB)
  def _(r):
    @pl.loop(0, d // 16)
    def _(c):
      rows0[r, pl.ds(c * 16, 16)] = jnp.zeros((16,), jnp.float32)
  off = 0
  while off < rows_per_tile:
    c = min(EB, rows_per_tile - off)
    pltpu.sync_copy(rows0.at[pl.ds(0, c)], acc.at[pl.ds(base + off, c)])
    off += c


def _ring(ngroups, srcv, dstv, rows, gsem, ssem, gather_from, acc):
  def issue_gather(j, b):
    return pltpu.async_copy(gather_from.at[srcv.at[j]], rows[b], gsem[b])

  def issue_scatter(j, b):
    return pltpu.async_copy(rows[b], acc.at[dstv.at[j]], ssem[b], add=True)

  gds = [issue_gather(b, b) for b in range(NBUF)]

  @pl.loop(0, ngroups - 1)
  def _(g):
    sds = []
    for b in range(NBUF):
      gds[b].wait()
      sds.append(issue_scatter(g * NBUF + b, b))
    for b in range(NBUF):
      sds[b].wait()
      issue_gather((g + 1) * NBUF + b, b)

  last = ngroups - 1
  sds = []
  for b in range(NBUF):
    gds[b].wait()
    sds.append(issue_scatter(last * NBUF + b, b))
  for b in range(NBUF):
    sds[b].wait()


def _make_prop0():
  d = 32
  nbs = NBS0
  ngroups = nbs // NBUF
  rows_per_tile = NP0 // NS

  def body(xs_hbm, src_hbm, dst_hbm, out_hbm, srcv, dstv, *rest):
    rows = list(rest[0:NBUF])
    table = rest[NBUF]
    acc = rest[NBUF + 1]
    gsem = list(rest[NBUF + 2: NBUF + 2 + NBUF])
    ssem = list(rest[NBUF + 2 + NBUF:])
    cid = lax.axis_index("c")
    sid = lax.axis_index("s")

    pltpu.sync_copy(dst_hbm.at[sid], dstv)
    base = pl.multiple_of(sid * rows_per_tile, 8)

    for p in range(2):
      q = 2 * cid + p
      pltpu.sync_copy(src_hbm.at[q * NS + sid], srcv)
      hoff = pl.multiple_of(q * NP0 + sid * rows_per_tile, 8)
      pltpu.sync_copy(xs_hbm.at[pl.ds(hoff, rows_per_tile)],
                      table.at[pl.ds(base, rows_per_tile)])
      _zero_acc_slice(rows[0], acc, d, base, rows_per_tile)
      plsc.subcore_barrier()

      _ring(ngroups, srcv, dstv, rows, gsem, ssem, table, acc)
      plsc.subcore_barrier()

      off = 0
      while off < rows_per_tile:
        c = min(512, rows_per_tile - off)
        pltpu.sync_copy(acc.at[pl.ds(base + off, c)],
                        out_hbm.at[q, pl.ds(base + off, c)])
        off += c

  return pl.kernel(
      body,
      out_type=jax.ShapeDtypeStruct((4, NP0, d), jnp.float32),
      mesh=_MESH,
      compiler_params=pltpu.CompilerParams(use_tc_tiling_on_sc=False),
      scratch_types=(
          [pltpu.VMEM((nbs, EB), jnp.int32) for _ in range(2)]
          + [pltpu.VMEM((EB, d), jnp.float32) for _ in range(NBUF)]
          + [pltpu.VMEM_SHARED((NP0, d), jnp.float32)]
          + [pltpu.VMEM_SHARED((NP0, d), jnp.float32)]
          + [pltpu.SemaphoreType.DMA for _ in range(2 * NBUF)]))


def _make_prop0_r2():
  d = 64
  nbs = NBS0
  ngroups = nbs // NBUF
  rows_per_tile = NP0 // NS

  def body(xs_hbm, src_hbm, dst_hbm, out_hbm, srcv, dstv, *rest):
    rows = list(rest[0:NBUF])
    acc = rest[NBUF]
    gsem = list(rest[NBUF + 1: NBUF + 1 + NBUF])
    ssem = list(rest[NBUF + 1 + NBUF:])
    cid = lax.axis_index("c")
    sid = lax.axis_index("s")

    pltpu.sync_copy(src_hbm.at[cid * NS + sid], srcv)
    pltpu.sync_copy(dst_hbm.at[sid], dstv)

    base = pl.multiple_of(sid * rows_per_tile, 8)
    _zero_acc_slice(rows[0], acc, d, base, rows_per_tile)
    plsc.subcore_barrier()

    _ring(ngroups, srcv, dstv, rows, gsem, ssem, xs_hbm, acc)
    plsc.subcore_barrier()

    off = 0
    while off < rows_per_tile:
      c = min(512, rows_per_tile - off)
      pltpu.sync_copy(acc.at[pl.ds(base + off, c)],
                      out_hbm.at[cid, pl.ds(base + off, c)])
      off += c

  return pl.kernel(
      body,
      out_type=jax.ShapeDtypeStruct((NC, NP0, d), jnp.float32),
      mesh=_MESH,
      compiler_params=pltpu.CompilerParams(use_tc_tiling_on_sc=False),
      scratch_types=(
          [pltpu.VMEM((nbs, EB), jnp.int32) for _ in range(2)]
          + [pltpu.VMEM((EB, d), jnp.float32) for _ in range(NBUF)]
          + [pltpu.VMEM_SHARED((NP0, d), jnp.float32)]
          + [pltpu.SemaphoreType.DMA for _ in range(2 * NBUF)]))


def _make_prop1():
  d = 64
  nbs = NBS1
  ngroups = nbs // NBUF
  rows_per_tile = NP1 // NS

  def body(xs_hbm, snd_hbm, src_hbm, dst_hbm, out_hbm, srcv, dstv, *rest):
    rows = list(rest[0:NBUF])
    mbuf = rest[NBUF]
    sndv = rest[NBUF + 1]
    table = rest[NBUF + 2]
    acc = rest[NBUF + 3]
    gsem = list(rest[NBUF + 4: NBUF + 4 + NBUF])
    ssem = list(rest[NBUF + 4 + NBUF:])
    cid = lax.axis_index("c")
    sid = lax.axis_index("s")

    pltpu.sync_copy(src_hbm.at[sid], srcv)
    pltpu.sync_copy(dst_hbm.at[sid], dstv)

    base = pl.multiple_of(sid * rows_per_tile, 8)
    hoff = pl.multiple_of(cid * NP1 + sid * rows_per_tile, 8)
    pltpu.sync_copy(xs_hbm.at[pl.ds(hoff, rows_per_tile)],
                    table.at[pl.ds(base, rows_per_tile)])
    pltpu.sync_copy(snd_hbm.at[pl.ds(base, rows_per_tile)], sndv)
    _zero_acc_slice(rows[0], acc, d, base, rows_per_tile)
    plsc.subcore_barrier()

    _ring(ngroups, srcv, dstv, rows, gsem, ssem, table, acc)
    plsc.subcore_barrier()

    pltpu.sync_copy(acc.at[pl.ds(base, rows_per_tile)], mbuf)
    @pl.loop(0, rows_per_tile)
    def _(r):
      s = sndv[r, pl.ds(0, 16)]
      for c in range(d // 16):
        mbuf[r, pl.ds(c * 16, 16)] = mbuf[r, pl.ds(c * 16, 16)] * s
    pltpu.sync_copy(mbuf, table.at[pl.ds(base, rows_per_tile)])
    _zero_acc_slice(rows[0], acc, d, base, rows_per_tile)
    plsc.subcore_barrier()

    _ring(ngroups, srcv, dstv, rows, gsem, ssem, table, acc)
    plsc.subcore_barrier()

    off = 0
    while off < rows_per_tile:
      c = min(512, rows_per_tile - off)
      pltpu.sync_copy(acc.at[pl.ds(base + off, c)],
                      out_hbm.at[cid, pl.ds(base + off, c)])
      off += c

  return pl.kernel(
      body,
      out_type=jax.ShapeDtypeStruct((NC, NP1, d), jnp.float32),
      mesh=_MESH,
      compiler_params=pltpu.CompilerParams(use_tc_tiling_on_sc=False),
      scratch_types=(
          [pltpu.VMEM((nbs, EB), jnp.int32) for _ in range(2)]
          + [pltpu.VMEM((EB, d), jnp.float32) for _ in range(NBUF)]
          + [pltpu.VMEM((NP1 // NS, d), jnp.float32)]
          + [pltpu.VMEM((NP1 // NS, 16), jnp.float32)]
          + [pltpu.VMEM_SHARED((NP1, d), jnp.float32)]
          + [pltpu.VMEM_SHARED((NP1, d), jnp.float32)]
          + [pltpu.SemaphoreType.DMA for _ in range(2 * NBUF)]))


def _make_prop2():
  d = 64
  nb = NB2
  ngroups = nb // NBUF
  rows_per_tile = NP2 // NS

  def body(xs_hbm, src_hbm, dst_hbm, out_hbm, srcv, dstv, *rest):
    rows = list(rest[0:NBUF])
    table = rest[NBUF]
    acc = rest[NBUF + 1]
    gsem = list(rest[NBUF + 2: NBUF + 2 + NBUF])
    ssem = list(rest[NBUF + 2 + NBUF:])
    cid = lax.axis_index("c")
    sid = lax.axis_index("s")
    wid = sid * NC + cid

    pltpu.sync_copy(src_hbm.at[wid], srcv)
    pltpu.sync_copy(dst_hbm.at[wid], dstv)

    base = pl.multiple_of(sid * rows_per_tile, 8)
    pltpu.sync_copy(xs_hbm.at[pl.ds(base, rows_per_tile)],
                    table.at[pl.ds(base, rows_per_tile)])
    _zero_acc_slice(rows[0], acc, d, base, rows_per_tile)
    plsc.subcore_barrier()

    _ring(ngroups, srcv, dstv, rows, gsem, ssem, table, acc)
    plsc.subcore_barrier()

    off = 0
    while off < rows_per_tile:
      c = min(512, rows_per_tile - off)
      pltpu.sync_copy(acc.at[pl.ds(base + off, c)],
                      out_hbm.at[cid, pl.ds(base + off, c)])
      off += c

  return pl.kernel(
      body,
      out_type=jax.ShapeDtypeStruct((NC, NP2, d), jnp.float32),
      mesh=_MESH,
      compiler_params=pltpu.CompilerParams(use_tc_tiling_on_sc=False),
      scratch_types=(
          [pltpu.VMEM((nb, EB), jnp.int32) for _ in range(2)]
          + [pltpu.VMEM((EB, d), jnp.float32) for _ in range(NBUF)]
          + [pltpu.VMEM_SHARED((NP2, d), jnp.float32)]
          + [pltpu.VMEM_SHARED((NP2, d), jnp.float32)]
          + [pltpu.SemaphoreType.DMA for _ in range(2 * NBUF)]))


_prop0 = _make_prop0()
_prop0r2 = _make_prop0_r2()
_prop1 = _make_prop1()
_prop2 = _make_prop2()


def _tc_invs(a, b, c, d, n, blk):
  def body(a_r, b_r, c_r, d_r, ns_o, nd_o):
    def invs(x, y):
      dg = x[...] + y[...]
      return jnp.where(dg > 0, lax.rsqrt(dg), 0.0)
    ns_o[...] = invs(a_r, b_r)
    nd_o[...] = invs(c_r, d_r)

  spec = pl.BlockSpec((blk, 1), lambda i: (i, 0))
  return pl.pallas_call(
      body,
      grid=(n // blk,),
      in_specs=[spec, spec, spec, spec],
      out_specs=[spec, spec],
      out_shape=[jax.ShapeDtypeStruct((n, 1), jnp.float32),
                 jax.ShapeDtypeStruct((n, 1), jnp.float32)],
  )(a, b, c, d)


def _tc_scale0(featp, ns0):
  blk = 1264

  def body(f_r, ns_r, o_r):
    xs = f_r[...] * ns_r[...]
    o_r[0] = xs[:, :64]
    o_r[1] = xs[:, 64:]

  return pl.pallas_call(
      body,
      grid=(NP0 // blk,),
      in_specs=[pl.BlockSpec((blk, 128), lambda i: (i, 0)),
                pl.BlockSpec((blk, 1), lambda i: (i, 0))],
      out_specs=pl.BlockSpec((2, blk, 64), lambda i: (0, i, 0)),
      out_shape=jax.ShapeDtypeStruct((2, NP0, 64), jnp.float32),
  )(featp, ns0)


def _tc_layer0(p0h, nd0, W1, b1r):
  def body(plo_r, phi_r, nd_r, w_r, b_r, z_r):
    nd = nd_r[...]
    acc = b_r[...] + jnp.dot(plo_r[...][0] * nd, w_r[:64, :],
                             preferred_element_type=jnp.float32)
    acc += jnp.dot(phi_r[...][0] * nd, w_r[64:, :],
                   preferred_element_type=jnp.float32)
    z_r[...] = jnp.maximum(acc, 0.0)

  return pl.pallas_call(
      body,
      grid=(10,),
      in_specs=[pl.BlockSpec((1, 1000, 64), lambda i: (0, i, 0)),
                pl.BlockSpec((1, 1000, 64), lambda i: (1, i, 0)),
                pl.BlockSpec((1000, 1), lambda i: (i, 0)),
                pl.BlockSpec((128, 128), lambda i: (0, 0)),
                pl.BlockSpec((1, 128), lambda i: (0, 0))],
      out_specs=pl.BlockSpec((1000, 128), lambda i: (i, 0)),
      out_shape=jax.ShapeDtypeStruct((N0, 128), jnp.float32),
  )(p0h, p0h, nd0, W1, b1r)


def _tc_snd16(ns1, nd1):
  blk = 640

  def body(ns_r, nd_r, o_r):
    i = pl.program_id(0)
    row = i * blk + lax.broadcasted_iota(jnp.int32, (blk, 16), 0)
    s = ns_r[...] * nd_r[...]
    o_r[...] = jnp.where(row < N1, s, 0.0)

  return pl.pallas_call(
      body,
      grid=(NP1 // blk,),
      in_specs=[pl.BlockSpec((blk, 1), lambda i: (i, 0)),
                pl.BlockSpec((blk, 1), lambda i: (i, 0))],
      out_specs=pl.BlockSpec((blk, 16), lambda i: (i, 0)),
      out_shape=jax.ShapeDtypeStruct((NP1, 16), jnp.float32),
  )(ns1, nd1)


def _tc_proj0(z, P0T, ns1):
  blk = 256

  def body(p_r, z_r, ns_r, emb_r, xt_r):
    part = lax.dot_general(p_r[...], z_r[...], (((1,), (0,)), ((), ())),
                           preferred_element_type=jnp.float32)
    emb_r[...] = part
    xs = part * ns_r[...]
    xt_r[0] = xs[:, :64]
    xt_r[1] = xs[:, 64:]

  return pl.pallas_call(
      body,
      grid=(20,),
      in_specs=[pl.BlockSpec((blk, N0), lambda j: (j, 0)),
                pl.BlockSpec((N0, 128), lambda j: (0, 0)),
                pl.BlockSpec((blk, 1), lambda j: (j, 0))],
      out_specs=[pl.BlockSpec((blk, 128), lambda j: (j, 0)),
                 pl.BlockSpec((2, blk, 64), lambda j: (0, j, 0))],
      out_shape=[jax.ShapeDtypeStruct((N1, 128), jnp.float32),
                 jax.ShapeDtypeStruct((2, NP1, 64), jnp.float32)],
  )(P0T, z, ns1)


def _tc_proj1(part1b, nd1, P1, W2, ns2):
  nk = 5

  def body(p_r, plo_r, phi_r, nd_r, w2_r, ns_r, x2_r, acc_r):
    k = pl.program_id(1)
    h1 = jnp.concatenate([plo_r[...][0], phi_r[...][0]], axis=1) * nd_r[...]
    part = lax.dot_general(p_r[...], h1, (((0,), (0,)), ((), ())),
                           preferred_element_type=jnp.float32)
    @pl.when(k == 0)
    def _():
      acc_r[...] = part
    @pl.when(k > 0)
    def _():
      acc_r[...] += part
    @pl.when(k == nk - 1)
    def _():
      x2_r[...] = jnp.dot(acc_r[...], w2_r[...],
                          preferred_element_type=jnp.float32) * ns_r[...]

  return pl.pallas_call(
      body,
      grid=(5, nk),
      in_specs=[pl.BlockSpec((1000, 512), lambda j, k: (k, j)),
                pl.BlockSpec((1, 1000, 64), lambda j, k: (0, k, 0)),
                pl.BlockSpec((1, 1000, 64), lambda j, k: (1, k, 0)),
                pl.BlockSpec((1000, 1), lambda j, k: (k, 0)),
                pl.BlockSpec((128, 40), lambda j, k: (0, 0)),
                pl.BlockSpec((512, 1), lambda j, k: (j, 0))],
      out_specs=pl.BlockSpec((512, 40), lambda j, k: (j, 0)),
      out_shape=jax.ShapeDtypeStruct((N2, 40), jnp.float32),
      scratch_shapes=[pltpu.VMEM((512, 128), jnp.float32)],
  )(P1, part1b, part1b, nd1, W2, ns2)


def _tc_final(pa, pb, nd2, b2r):
  def body(pa_r, pb_r, nd_r, b_r, o_r):
    o_r[...] = (pa_r[...] + pb_r[...]) * nd_r[...] + b_r[...]

  return pl.pallas_call(
      body,
      out_shape=jax.ShapeDtypeStruct((N2, 40), jnp.float32),
  )(pa, pb, nd2, b2r)


def _prep_edges(ei, nb, dummy):
  e_pad = NW * nb * EB
  pad = e_pad - ei.shape[1]
  fill = jnp.full((pad,), dummy, dtype=jnp.int32)
  src = jnp.concatenate([ei[0], fill]).reshape(NW, nb, EB)
  dst = jnp.concatenate([ei[1], fill]).reshape(NW, nb, EB)
  return src, dst


def kernel(features, edge_index0, edge_index1, edge_index2, P0, P1, W1, b1,
           W2, b2):
  s0, d0 = _prep_edges(edge_index0, NB0, N0)
  s1, d1 = _prep_edges(edge_index1, NB1, N1)
  s2, d2 = _prep_edges(edge_index2, NB2, N2)

  dg0s, dg0d, dg1s, dg1d, dg2s, dg2d = _deg_kernel(s0, d0, s1, d1, s2, d2)

  s0s = s0.reshape(NS, NBS0, EB)
  s0c = jnp.concatenate([s0s + q * NP0 for q in range(4)], axis=0)
  d0c = d0.reshape(NS, NBS0, EB)
  s1c = s1.reshape(NS, NBS1, EB)
  d1c = d1.reshape(NS, NBS1, EB)

  featp = jnp.pad(features, ((0, NP0 - N0), (0, 0)))

  def cut(dg, size, npad):
    return dg[:npad, None], dg[size:size + npad, None]

  a0s, b0s = cut(dg0s, S0, NP0)
  a0d, b0d = cut(dg0d, S0, NP0)
  a1s, b1s = cut(dg1s, S1, NP1)
  a1d, b1d = cut(dg1d, S1, NP1)
  a2s, b2s = cut(dg2s, S2, NP2)
  a2d, b2d = cut(dg2d, S2, NP2)

  ns0, nd0 = _tc_invs(a0s, b0s, a0d, b0d, NP0, 1264)
  ns1, nd1 = _tc_invs(a1s, b1s, a1d, b1d, NP1, 640)
  ns2, nd2 = _tc_invs(a2s, b2s, a2d, b2d, NP2, 320)
  xt0 = _tc_scale0(featp, ns0)
  s0c2 = jnp.concatenate([s0s, s0s + NP0], axis=0)

  p0h = _prop0r2(xt0.reshape(2 * NP0, 64), s0c2, d0c)
  z = _tc_layer0(p0h, nd0, W1, b1.reshape(1, 128))

  emb, xt1 = _tc_proj0(z, P0.T, ns1)
  snd16 = _tc_snd16(ns1, nd1)

  part1b = _prop1(xt1.reshape(2 * NP1, 64), snd16, s1c, d1c)
  x2s = _tc_proj1(part1b, nd1, P1, W2, ns2[:N2])

  xt2 = jnp.pad(x2s, ((0, NP2 - N2), (0, 24)))
  part2 = _prop2(xt2, s2, d2)
  out = _tc_final(part2[0, :N2, :40], part2[1, :N2, :40], nd2[:N2],
                  b2.reshape(1, 40))
  return (out, emb)


# speedup vs baseline: 9.9657x; 1.0001x over previous
import jax
import jax.numpy as jnp
from jax import lax
from jax.experimental import pallas as pl
from jax.experimental.pallas import tpu as pltpu
from jax.experimental.pallas import tpu_sc as plsc

NC = 2
NS = 16
NW = NC * NS
EB = 128
NBUF = 4

N0, N1, N2 = 10000, 5000, 2500
NP0, NP1, NP2 = 10112, 5120, 2560
NB0, NB1, NB2 = 80, 40, 20
NBS0, NBS1 = 160, 80
S0, S1, S2 = 10240, 6144, 4096

_MESH = plsc.VectorSubcoreMesh(
    core_axis_name="c", subcore_axis_name="s", num_cores=NC, num_subcores=NS)


def _zero_vec(ref, n):
  @pl.loop(0, n // 16)
  def _(i):
    ref[pl.ds(i * 16, 16)] = jnp.zeros((16,), jnp.float32)


def _deg_body(e0s, e0d, e1s, e1d, e2s, e2d,
              o0s, o0d, o1s, o1d, o2s, o2d,
              idxv, onesv, zbuf,
              sec0s, sec0d, sec1s, sec1d, sec2s, sec2d, dsem):
  cid = lax.axis_index("c")
  sid = lax.axis_index("s")
  wid = sid * NC + cid

  _zero_vec(zbuf, 640)
  @pl.loop(0, EB // 16)
  def _(i):
    onesv[pl.ds(i * 16, 16)] = jnp.ones((16,), jnp.float32)

  passes = [(e0s, sec0s, o0s, S0, NB0), (e0d, sec0d, o0d, S0, NB0),
            (e1s, sec1s, o1s, S1, NB1), (e1d, sec1d, o1d, S1, NB1),
            (e2s, sec2s, o2s, S2, NB2), (e2d, sec2d, o2d, S2, NB2)]

  for _, sec, _, size, _ in passes:
    sz = size // NS
    pltpu.sync_copy(zbuf.at[pl.ds(0, sz)], sec.at[pl.ds(sid * sz, sz)])
  plsc.subcore_barrier()

  for e_ref, sec, _, _, nb in passes:
    pltpu.sync_copy(e_ref.at[wid], idxv.at[pl.ds(0, nb)])
    @pl.loop(0, nb // 4)
    def _(g):
      descs = []
      for b in range(4):
        descs.append(
            pltpu.async_copy(onesv, sec.at[idxv.at[g * 4 + b]], dsem,
                             add=True))
      for d_ in descs:
        d_.wait()
  plsc.subcore_barrier()

  for _, sec, out, size, _ in passes:
    sz = size // NS
    off = pl.multiple_of(cid * size + sid * sz, 128)
    pltpu.sync_copy(sec.at[pl.ds(sid * sz, sz)], out.at[pl.ds(off, sz)])


_deg_kernel = pl.kernel(
    _deg_body,
    out_type=[jax.ShapeDtypeStruct((NC * S0,), jnp.float32),
              jax.ShapeDtypeStruct((NC * S0,), jnp.float32),
              jax.ShapeDtypeStruct((NC * S1,), jnp.float32),
              jax.ShapeDtypeStruct((NC * S1,), jnp.float32),
              jax.ShapeDtypeStruct((NC * S2,), jnp.float32),
              jax.ShapeDtypeStruct((NC * S2,), jnp.float32)],
    mesh=_MESH,
    scratch_types=[
        pltpu.VMEM((NB0, EB), jnp.int32),
        pltpu.VMEM((EB,), jnp.float32),
        pltpu.VMEM((640,), jnp.float32),
        pltpu.VMEM_SHARED((S0,), jnp.float32),
        pltpu.VMEM_SHARED((S0,), jnp.float32),
        pltpu.VMEM_SHARED((S1,), jnp.float32),
        pltpu.VMEM_SHARED((S1,), jnp.float32),
        pltpu.VMEM_SHARED((S2,), jnp.float32),
        pltpu.VMEM_SHARED((S2,), jnp.float32),
        pltpu.SemaphoreType.DMA,
    ])


def _zero_acc_slice(rows0, acc, d, base, rows_per_tile):
  @pl.loop(0, EB)
  def _(r):
    @pl.loop(0, d // 16)
    def _(c):
      rows0[r, pl.ds(c * 16, 16)] = jnp.zeros((16,), jnp.float32)
  off = 0
  while off < rows_per_tile:
    c = min(EB, rows_per_tile - off)
    pltpu.sync_copy(rows0.at[pl.ds(0, c)], acc.at[pl.ds(base + off, c)])
    off += c


def _ring(ngroups, srcv, dstv, rows, gsem, ssem, gather_from, acc):
  def issue_gather(j, b):
    return pltpu.async_copy(gather_from.at[srcv.at[j]], rows[b], gsem[b])

  def issue_scatter(j, b):
    return pltpu.async_copy(rows[b], acc.at[dstv.at[j]], ssem[b], add=True)

  gds = [issue_gather(b, b) for b in range(NBUF)]

  @pl.loop(0, ngroups - 1)
  def _(g):
    sds = []
    for b in range(NBUF):
      gds[b].wait()
      sds.append(issue_scatter(g * NBUF + b, b))
    for b in range(NBUF):
      sds[b].wait()
      issue_gather((g + 1) * NBUF + b, b)

  last = ngroups - 1
  sds = []
  for b in range(NBUF):
    gds[b].wait()
    sds.append(issue_scatter(last * NBUF + b, b))
  for b in range(NBUF):
    sds[b].wait()


def _make_prop0():
  d = 64
  nbs = NBS0
  ngroups = nbs // NBUF
  rows_per_tile = NP0 // NS

  def body(xs_hbm, src_hbm, dst_hbm, out_hbm, srcv, dstv, *rest):
    rows = list(rest[0:NBUF])
    acc = rest[NBUF]
    gsem = list(rest[NBUF + 1: NBUF + 1 + NBUF])
    ssem = list(rest[NBUF + 1 + NBUF:])
    cid = lax.axis_index("c")
    sid = lax.axis_index("s")

    pltpu.sync_copy(src_hbm.at[cid * NS + sid], srcv)
    pltpu.sync_copy(dst_hbm.at[sid], dstv)

    base = pl.multiple_of(sid * rows_per_tile, 8)
    _zero_acc_slice(rows[0], acc, d, base, rows_per_tile)
    plsc.subcore_barrier()

    _ring(ngroups, srcv, dstv, rows, gsem, ssem, xs_hbm, acc)
    plsc.subcore_barrier()

    off = 0
    while off < rows_per_tile:
      c = min(512, rows_per_tile - off)
      pltpu.sync_copy(acc.at[pl.ds(base + off, c)],
                      out_hbm.at[cid, pl.ds(base + off, c)])
      off += c

  return pl.kernel(
      body,
      out_type=jax.ShapeDtypeStruct((NC, NP0, d), jnp.float32),
      mesh=_MESH,
      compiler_params=pltpu.CompilerParams(use_tc_tiling_on_sc=False),
      scratch_types=(
          [pltpu.VMEM((nbs, EB), jnp.int32) for _ in range(2)]
          + [pltpu.VMEM((EB, d), jnp.float32) for _ in range(NBUF)]
          + [pltpu.VMEM_SHARED((NP0, d), jnp.float32)]
          + [pltpu.SemaphoreType.DMA for _ in range(2 * NBUF)]))


def _make_prop1():
  d = 64
  nbs = NBS1
  ngroups = nbs // NBUF
  rows_per_tile = NP1 // NS

  def body(xs_hbm, snd_hbm, src_hbm, dst_hbm, out_hbm, srcv, dstv, *rest):
    rows = list(rest[0:NBUF])
    mbuf = rest[NBUF]
    sndv = rest[NBUF + 1]
    table = rest[NBUF + 2]
    acc = rest[NBUF + 3]
    gsem = list(rest[NBUF + 4: NBUF + 4 + NBUF])
    ssem = list(rest[NBUF + 4 + NBUF:])
    cid = lax.axis_index("c")
    sid = lax.axis_index("s")

    pltpu.sync_copy(src_hbm.at[sid], srcv)
    pltpu.sync_copy(dst_hbm.at[sid], dstv)

    base = pl.multiple_of(sid * rows_per_tile, 8)
    hoff = pl.multiple_of(cid * NP1 + sid * rows_per_tile, 8)
    pltpu.sync_copy(xs_hbm.at[pl.ds(hoff, rows_per_tile)],
                    table.at[pl.ds(base, rows_per_tile)])
    pltpu.sync_copy(snd_hbm.at[pl.ds(base, rows_per_tile)], sndv)
    _zero_acc_slice(rows[0], acc, d, base, rows_per_tile)
    plsc.subcore_barrier()

    _ring(ngroups, srcv, dstv, rows, gsem, ssem, table, acc)
    plsc.subcore_barrier()

    pltpu.sync_copy(acc.at[pl.ds(base, rows_per_tile)], mbuf)
    @pl.loop(0, rows_per_tile)
    def _(r):
      s = sndv[r, pl.ds(0, 16)]
      for c in range(d // 16):
        mbuf[r, pl.ds(c * 16, 16)] = mbuf[r, pl.ds(c * 16, 16)] * s
    pltpu.sync_copy(mbuf, table.at[pl.ds(base, rows_per_tile)])
    _zero_acc_slice(rows[0], acc, d, base, rows_per_tile)
    plsc.subcore_barrier()

    _ring(ngroups, srcv, dstv, rows, gsem, ssem, table, acc)
    plsc.subcore_barrier()

    off = 0
    while off < rows_per_tile:
      c = min(512, rows_per_tile - off)
      pltpu.sync_copy(acc.at[pl.ds(base + off, c)],
                      out_hbm.at[cid, pl.ds(base + off, c)])
      off += c

  return pl.kernel(
      body,
      out_type=jax.ShapeDtypeStruct((NC, NP1, d), jnp.float32),
      mesh=_MESH,
      compiler_params=pltpu.CompilerParams(use_tc_tiling_on_sc=False),
      scratch_types=(
          [pltpu.VMEM((nbs, EB), jnp.int32) for _ in range(2)]
          + [pltpu.VMEM((EB, d), jnp.float32) for _ in range(NBUF)]
          + [pltpu.VMEM((NP1 // NS, d), jnp.float32)]
          + [pltpu.VMEM((NP1 // NS, 16), jnp.float32)]
          + [pltpu.VMEM_SHARED((NP1, d), jnp.float32)]
          + [pltpu.VMEM_SHARED((NP1, d), jnp.float32)]
          + [pltpu.SemaphoreType.DMA for _ in range(2 * NBUF)]))


def _make_prop2():
  d = 64
  nb = NB2
  ngroups = nb // NBUF
  rows_per_tile = NP2 // NS

  def body(xs_hbm, src_hbm, dst_hbm, out_hbm, srcv, dstv, *rest):
    rows = list(rest[0:NBUF])
    table = rest[NBUF]
    acc = rest[NBUF + 1]
    gsem = list(rest[NBUF + 2: NBUF + 2 + NBUF])
    ssem = list(rest[NBUF + 2 + NBUF:])
    cid = lax.axis_index("c")
    sid = lax.axis_index("s")
    wid = sid * NC + cid

    pltpu.sync_copy(src_hbm.at[wid], srcv)
    pltpu.sync_copy(dst_hbm.at[wid], dstv)

    base = pl.multiple_of(sid * rows_per_tile, 8)
    pltpu.sync_copy(xs_hbm.at[pl.ds(base, rows_per_tile)],
                    table.at[pl.ds(base, rows_per_tile)])
    _zero_acc_slice(rows[0], acc, d, base, rows_per_tile)
    plsc.subcore_barrier()

    _ring(ngroups, srcv, dstv, rows, gsem, ssem, table, acc)
    plsc.subcore_barrier()

    off = 0
    while off < rows_per_tile:
      c = min(512, rows_per_tile - off)
      pltpu.sync_copy(acc.at[pl.ds(base + off, c)],
                      out_hbm.at[cid, pl.ds(base + off, c)])
      off += c

  return pl.kernel(
      body,
      out_type=jax.ShapeDtypeStruct((NC, NP2, d), jnp.float32),
      mesh=_MESH,
      compiler_params=pltpu.CompilerParams(use_tc_tiling_on_sc=False),
      scratch_types=(
          [pltpu.VMEM((nb, EB), jnp.int32) for _ in range(2)]
          + [pltpu.VMEM((EB, d), jnp.float32) for _ in range(NBUF)]
          + [pltpu.VMEM_SHARED((NP2, d), jnp.float32)]
          + [pltpu.VMEM_SHARED((NP2, d), jnp.float32)]
          + [pltpu.SemaphoreType.DMA for _ in range(2 * NBUF)]))


_prop0 = _make_prop0()
_prop1 = _make_prop1()
_prop2 = _make_prop2()


def _tc_invs(a, b, c, d, n, blk):
  def body(a_r, b_r, c_r, d_r, ns_o, nd_o):
    def invs(x, y):
      dg = x[...] + y[...]
      return jnp.where(dg > 0, lax.rsqrt(dg), 0.0)
    ns_o[...] = invs(a_r, b_r)
    nd_o[...] = invs(c_r, d_r)

  spec = pl.BlockSpec((blk, 1), lambda i: (i, 0))
  return pl.pallas_call(
      body,
      grid=(n // blk,),
      in_specs=[spec, spec, spec, spec],
      out_specs=[spec, spec],
      out_shape=[jax.ShapeDtypeStruct((n, 1), jnp.float32),
                 jax.ShapeDtypeStruct((n, 1), jnp.float32)],
  )(a, b, c, d)


def _tc_scale0(featp, ns0):
  blk = 1264

  def body(f_r, ns_r, o_r):
    xs = f_r[...] * ns_r[...]
    o_r[0] = xs[:, :64]
    o_r[1] = xs[:, 64:]

  return pl.pallas_call(
      body,
      grid=(NP0 // blk,),
      in_specs=[pl.BlockSpec((blk, 128), lambda i: (i, 0)),
                pl.BlockSpec((blk, 1), lambda i: (i, 0))],
      out_specs=pl.BlockSpec((2, blk, 64), lambda i: (0, i, 0)),
      out_shape=jax.ShapeDtypeStruct((2, NP0, 64), jnp.float32),
  )(featp, ns0)


def _tc_layer0(p0h, nd0, W1, b1r):
  def body(plo_r, phi_r, nd_r, w_r, b_r, z_r):
    nd = nd_r[...]
    acc = b_r[...] + jnp.dot(plo_r[...][0] * nd, w_r[:64, :],
                             preferred_element_type=jnp.float32)
    acc += jnp.dot(phi_r[...][0] * nd, w_r[64:, :],
                   preferred_element_type=jnp.float32)
    z_r[...] = jnp.maximum(acc, 0.0)

  return pl.pallas_call(
      body,
      grid=(10,),
      in_specs=[pl.BlockSpec((1, 1000, 64), lambda i: (0, i, 0)),
                pl.BlockSpec((1, 1000, 64), lambda i: (1, i, 0)),
                pl.BlockSpec((1000, 1), lambda i: (i, 0)),
                pl.BlockSpec((128, 128), lambda i: (0, 0)),
                pl.BlockSpec((1, 128), lambda i: (0, 0))],
      out_specs=pl.BlockSpec((1000, 128), lambda i: (i, 0)),
      out_shape=jax.ShapeDtypeStruct((N0, 128), jnp.float32),
  )(p0h, p0h, nd0, W1, b1r)


def _tc_snd16(ns1, nd1):
  blk = 640

  def body(ns_r, nd_r, o_r):
    i = pl.program_id(0)
    row = i * blk + lax.broadcasted_iota(jnp.int32, (blk, 16), 0)
    s = ns_r[...] * nd_r[...]
    o_r[...] = jnp.where(row < N1, s, 0.0)

  return pl.pallas_call(
      body,
      grid=(NP1 // blk,),
      in_specs=[pl.BlockSpec((blk, 1), lambda i: (i, 0)),
                pl.BlockSpec((blk, 1), lambda i: (i, 0))],
      out_specs=pl.BlockSpec((blk, 16), lambda i: (i, 0)),
      out_shape=jax.ShapeDtypeStruct((NP1, 16), jnp.float32),
  )(ns1, nd1)


def _tc_proj0(z, P0T, ns1):
  blk = 256

  def body(p_r, z_r, ns_r, emb_r, xt_r):
    part = lax.dot_general(p_r[...], z_r[...], (((1,), (0,)), ((), ())),
                           preferred_element_type=jnp.float32)
    emb_r[...] = part
    xs = part * ns_r[...]
    xt_r[0] = xs[:, :64]
    xt_r[1] = xs[:, 64:]

  return pl.pallas_call(
      body,
      grid=(20,),
      in_specs=[pl.BlockSpec((blk, N0), lambda j: (j, 0)),
                pl.BlockSpec((N0, 128), lambda j: (0, 0)),
                pl.BlockSpec((blk, 1), lambda j: (j, 0))],
      out_specs=[pl.BlockSpec((blk, 128), lambda j: (j, 0)),
                 pl.BlockSpec((2, blk, 64), lambda j: (0, j, 0))],
      out_shape=[jax.ShapeDtypeStruct((N1, 128), jnp.float32),
                 jax.ShapeDtypeStruct((2, NP1, 64), jnp.float32)],
  )(P0T, z, ns1)


def _tc_proj1(part1b, nd1, P1, W2, ns2):
  nk = 5

  def body(p_r, plo_r, phi_r, nd_r, w2_r, ns_r, x2_r, acc_r):
    k = pl.program_id(1)
    h1 = jnp.concatenate([plo_r[...][0], phi_r[...][0]], axis=1) * nd_r[...]
    part = lax.dot_general(p_r[...], h1, (((0,), (0,)), ((), ())),
                           preferred_element_type=jnp.float32)
    @pl.when(k == 0)
    def _():
      acc_r[...] = part
    @pl.when(k > 0)
    def _():
      acc_r[...] += part
    @pl.when(k == nk - 1)
    def _():
      x2_r[...] = jnp.dot(acc_r[...], w2_r[...],
                          preferred_element_type=jnp.float32) * ns_r[...]

  return pl.pallas_call(
      body,
      grid=(5, nk),
      in_specs=[pl.BlockSpec((1000, 512), lambda j, k: (k, j)),
                pl.BlockSpec((1, 1000, 64), lambda j, k: (0, k, 0)),
                pl.BlockSpec((1, 1000, 64), lambda j, k: (1, k, 0)),
                pl.BlockSpec((1000, 1), lambda j, k: (k, 0)),
                pl.BlockSpec((128, 40), lambda j, k: (0, 0)),
                pl.BlockSpec((512, 1), lambda j, k: (j, 0))],
      out_specs=pl.BlockSpec((512, 40), lambda j, k: (j, 0)),
      out_shape=jax.ShapeDtypeStruct((N2, 40), jnp.float32),
      scratch_shapes=[pltpu.VMEM((512, 128), jnp.float32)],
  )(P1, part1b, part1b, nd1, W2, ns2)


def _tc_final(pa, pb, nd2, b2r):
  def body(pa_r, pb_r, nd_r, b_r, o_r):
    o_r[...] = (pa_r[...] + pb_r[...]) * nd_r[...] + b_r[...]

  return pl.pallas_call(
      body,
      out_shape=jax.ShapeDtypeStruct((N2, 40), jnp.float32),
  )(pa, pb, nd2, b2r)


def _prep_edges(ei, nb, dummy):
  e_pad = NW * nb * EB
  pad = e_pad - ei.shape[1]
  fill = jnp.full((pad,), dummy, dtype=jnp.int32)
  src = jnp.concatenate([ei[0], fill]).reshape(NW, nb, EB)
  dst = jnp.concatenate([ei[1], fill]).reshape(NW, nb, EB)
  return src, dst


def kernel(features, edge_index0, edge_index1, edge_index2, P0, P1, W1, b1,
           W2, b2):
  s0, d0 = _prep_edges(edge_index0, NB0, N0)
  s1, d1 = _prep_edges(edge_index1, NB1, N1)
  s2, d2 = _prep_edges(edge_index2, NB2, N2)

  dg0s, dg0d, dg1s, dg1d, dg2s, dg2d = _deg_kernel(s0, d0, s1, d1, s2, d2)

  s0s = s0.reshape(NS, NBS0, EB)
  d0c = d0.reshape(NS, NBS0, EB)
  s1c = s1.reshape(NS, NBS1, EB)
  d1c = d1.reshape(NS, NBS1, EB)

  featp = jnp.pad(features, ((0, NP0 - N0), (0, 0)))

  def cut(dg, size, npad):
    return dg[:npad, None], dg[size:size + npad, None]

  a0s, b0s = cut(dg0s, S0, NP0)
  a0d, b0d = cut(dg0d, S0, NP0)
  a1s, b1s = cut(dg1s, S1, NP1)
  a1d, b1d = cut(dg1d, S1, NP1)
  a2s, b2s = cut(dg2s, S2, NP2)
  a2d, b2d = cut(dg2d, S2, NP2)

  ns0, nd0 = _tc_invs(a0s, b0s, a0d, b0d, NP0, 1264)
  ns1, nd1 = _tc_invs(a1s, b1s, a1d, b1d, NP1, 640)
  ns2, nd2 = _tc_invs(a2s, b2s, a2d, b2d, NP2, 320)
  xt0 = _tc_scale0(featp, ns0)
  s0c2 = jnp.concatenate([s0s, s0s + NP0], axis=0)

  p0h = _prop0(xt0.reshape(2 * NP0, 64), s0c2, d0c)
  z = _tc_layer0(p0h, nd0, W1, b1.reshape(1, 128))

  emb, xt1 = _tc_proj0(z, P0.T, ns1)
  snd16 = _tc_snd16(ns1, nd1)

  part1b = _prop1(xt1.reshape(2 * NP1, 64), snd16, s1c, d1c)
  x2s = _tc_proj1(part1b, nd1, P1, W2, ns2[:N2])

  xt2 = jnp.pad(x2s, ((0, NP2 - N2), (0, 24)))
  part2 = _prop2(xt2, s2, d2)
  out = _tc_final(part2[0, :N2, :40], part2[1, :N2, :40], nd2[:N2],
                  b2.reshape(1, 40))
  return (out, emb)
